# Initial kernel scaffold; baseline (speedup 1.0000x reference)
import jax
import jax.numpy as jnp
from jax.experimental import pallas as pl


def kernel(event_type_ids, event_src_ids, event_dst_ids, event_embeddings, event_timestamps, x, edge_index, edge_attr, edge_last_update, batch, memory, emb_table, Wt, bt, W_ih, b_ih, W_hh, b_hh, Wq, Wk, Wv, We, Wskip, Wlin, blin):
    raise NotImplementedError("write your pallas kernel here")


# speedup vs baseline: 4.3615x; 4.3615x over previous
import functools

import jax
import jax.numpy as jnp
from jax import lax
from jax.experimental import pallas as pl
from jax.experimental.pallas import tpu as pltpu
from jax.experimental.pallas import tpu_sc as plsc

N = 10000
NP = 10112
E = 320000
B = 4096
NC = 2
NS = 16
NW = NC * NS

EV_CHUNK = 128
E_PER_TILE = E // NW
E_CHUNK = 40
R_CHUNK = 2000
SLABS = (128, 128, 128, 128, 120)

_mesh = plsc.VectorSubcoreMesh(core_axis_name="c", subcore_axis_name="s")
_cp = pltpu.CompilerParams(needs_layout_passes=False, use_tc_tiling_on_sc=False)


def _zero2d(ref, rows, cols):
    def body(i, _):
        for c in range(cols // 16):
            ref[i, pl.ds(c * 16, 16)] = jnp.zeros((16,), jnp.float32)
        return 0
    lax.fori_loop(0, rows, body, 0)


def _fill_table(zbuf, table, sid):
    for j, zr in enumerate(SLABS):
        pltpu.sync_copy(zbuf.at[pl.ds(0, zr)],
                        table.at[pl.ds(sid * 632 + j * 128, zr)])


def _dump_table(table, out, cid, sid):
    for j, zr in enumerate(SLABS):
        r0 = sid * 632 + j * 128
        pltpu.sync_copy(table.at[pl.ds(r0, zr)], out.at[cid, pl.ds(r0, zr)])


@functools.partial(
    pl.kernel, mesh=_mesh, compiler_params=_cp,
    out_type=[
        jax.ShapeDtypeStruct((NC, NP, 64), jnp.float32),
        jax.ShapeDtypeStruct((NC, NP, 64), jnp.float32),
    ],
    scratch_types=[
        pltpu.VMEM_SHARED((NP, 64), jnp.float32),
        pltpu.VMEM_SHARED((NP, 64), jnp.float32),
        pltpu.VMEM((EV_CHUNK, 64), jnp.float32),
        pltpu.VMEM((EV_CHUNK, 64), jnp.float32),
        pltpu.VMEM((EV_CHUNK,), jnp.int32),
        pltpu.VMEM((EV_CHUNK,), jnp.int32),
        pltpu.SemaphoreType.DMA,
        pltpu.SemaphoreType.DMA,
    ],
)
def _sc_a(src_ids, dst_ids, mem_hbm, o1, o2,
          s1, s2, smem_v, dmem_v, idxs_v, idxd_v, sem1, sem2):
    cid = lax.axis_index("c")
    sid = lax.axis_index("s")
    _zero2d(smem_v, EV_CHUNK, 64)
    _fill_table(smem_v, s1, sid)
    _fill_table(smem_v, s2, sid)
    plsc.subcore_barrier()

    ev0 = cid * (B // NC) + sid * EV_CHUNK
    pltpu.sync_copy(src_ids.at[pl.ds(ev0, EV_CHUNK)], idxs_v)
    pltpu.sync_copy(dst_ids.at[pl.ds(ev0, EV_CHUNK)], idxd_v)
    cp1 = pltpu.async_copy(mem_hbm.at[idxs_v], smem_v, sem1)
    cp2 = pltpu.async_copy(mem_hbm.at[idxd_v], dmem_v, sem2)
    cp1.wait()
    cp2.wait()
    pltpu.sync_copy(smem_v, s1.at[idxs_v], add=True)
    pltpu.sync_copy(smem_v, s1.at[idxd_v], add=True)
    pltpu.sync_copy(dmem_v, s2.at[idxs_v], add=True)
    pltpu.sync_copy(dmem_v, s2.at[idxd_v], add=True)

    plsc.subcore_barrier()
    _dump_table(s1, o1, cid, sid)
    _dump_table(s2, o2, cid, sid)


@functools.partial(
    pl.kernel, mesh=_mesh, compiler_params=_cp,
    out_type=[jax.ShapeDtypeStruct((NC, NP, 128), jnp.float32)],
    scratch_types=[
        pltpu.VMEM_SHARED((NP, 128), jnp.float32),
        pltpu.VMEM((EV_CHUNK, 128), jnp.float32),
        pltpu.VMEM((EV_CHUNK,), jnp.int32),
        pltpu.VMEM((EV_CHUNK,), jnp.int32),
    ],
)
def _sc_b(src_ids, dst_ids, eemb_hbm, o1, s1, ebuf_v, idxs_v, idxd_v):
    cid = lax.axis_index("c")
    sid = lax.axis_index("s")
    _zero2d(ebuf_v, EV_CHUNK, 128)
    _fill_table(ebuf_v, s1, sid)
    plsc.subcore_barrier()

    ev0 = cid * (B // NC) + sid * EV_CHUNK
    pltpu.sync_copy(src_ids.at[pl.ds(ev0, EV_CHUNK)], idxs_v)
    pltpu.sync_copy(dst_ids.at[pl.ds(ev0, EV_CHUNK)], idxd_v)
    pltpu.sync_copy(eemb_hbm.at[pl.ds(ev0, EV_CHUNK)], ebuf_v)
    pltpu.sync_copy(ebuf_v, s1.at[idxs_v], add=True)
    pltpu.sync_copy(ebuf_v, s1.at[idxd_v], add=True)

    plsc.subcore_barrier()
    _dump_table(s1, o1, cid, sid)


@functools.partial(
    pl.kernel, mesh=_mesh, compiler_params=_cp,
    out_type=[
        jax.ShapeDtypeStruct((NC, NP, 32), jnp.float32),
        jax.ShapeDtypeStruct((NC, NP, 16), jnp.float32),
        jax.ShapeDtypeStruct((E,), jnp.float32),
    ],
    scratch_types=[
        pltpu.VMEM_SHARED((NP, 32), jnp.float32),
        pltpu.VMEM_SHARED((NP, 16), jnp.float32),
        pltpu.VMEM((EV_CHUNK, 32), jnp.float32),
        pltpu.VMEM((EV_CHUNK, 16), jnp.float32),
        pltpu.VMEM((EV_CHUNK,), jnp.int32),
        pltpu.VMEM((EV_CHUNK,), jnp.int32),
        pltpu.VMEM((EV_CHUNK,), jnp.int32),
        pltpu.VMEM((112,), jnp.float32),
        pltpu.VMEM((B,), jnp.float32),
        pltpu.VMEM((R_CHUNK,), jnp.int32),
        pltpu.VMEM((N,), jnp.float32),
        pltpu.VMEM((R_CHUNK,), jnp.int32),
        pltpu.VMEM((R_CHUNK,), jnp.float32),
        pltpu.VMEM((R_CHUNK,), jnp.float32),
    ],
)
def _sc_c(src_ids, dst_ids, type_ids, time_embs, batch_hbm, ts_hbm,
          esrc_hbm, elu_hbm, embt_hbm,
          o1, o2, relt_out,
          s1, s2, tbuf_v, tybuf_v, idxs_v, idxd_v, tids_v, embt_v,
          tsv, bch_v, nts_v, sbuf_v, elub_v, rbuf_v):
    cid = lax.axis_index("c")
    sid = lax.axis_index("s")
    wid = sid * NC + cid
    _zero2d(tbuf_v, EV_CHUNK, 32)
    _fill_table(tbuf_v, s1, sid)
    _zero2d(tybuf_v, EV_CHUNK, 16)
    _fill_table(tybuf_v, s2, sid)
    plsc.subcore_barrier()

    pltpu.sync_copy(embt_hbm, embt_v)
    ev0 = cid * (B // NC) + sid * EV_CHUNK
    pltpu.sync_copy(src_ids.at[pl.ds(ev0, EV_CHUNK)], idxs_v)
    pltpu.sync_copy(dst_ids.at[pl.ds(ev0, EV_CHUNK)], idxd_v)
    pltpu.sync_copy(time_embs.at[pl.ds(ev0, EV_CHUNK)], tbuf_v)
    pltpu.sync_copy(type_ids.at[pl.ds(ev0, EV_CHUNK)], tids_v)
    lanes = lax.iota(jnp.int32, 16)

    def tyb(g, _):
        tv = tids_v[pl.ds(g * 16, 16)]
        for j in range(16):
            ti = tv[j]
            vals = plsc.load_gather(embt_v, [ti * 16 + lanes])
            tybuf_v[g * 16 + j, pl.ds(0, 16)] = vals
        return 0
    lax.fori_loop(0, EV_CHUNK // 16, tyb, 0)

    pltpu.sync_copy(tbuf_v, s1.at[idxs_v], add=True)
    pltpu.sync_copy(tbuf_v, s1.at[idxd_v], add=True)
    pltpu.sync_copy(tybuf_v, s2.at[idxs_v], add=True)
    pltpu.sync_copy(tybuf_v, s2.at[idxd_v], add=True)

    pltpu.sync_copy(ts_hbm, tsv)
    for m in range(N // R_CHUNK):
        pltpu.sync_copy(batch_hbm.at[pl.ds(m * R_CHUNK, R_CHUNK)], bch_v)

        def nbody(j, _):
            idx = bch_v[pl.ds(j * 16, 16)]
            nts_v[pl.ds(m * R_CHUNK + j * 16, 16)] = plsc.load_gather(tsv, [idx])
            return 0
        lax.fori_loop(0, R_CHUNK // 16, nbody, 0)

    for t in range(E_PER_TILE // R_CHUNK):
        eb = wid * E_PER_TILE + t * R_CHUNK
        pltpu.sync_copy(esrc_hbm.at[pl.ds(eb, R_CHUNK)], sbuf_v)
        pltpu.sync_copy(elu_hbm.at[pl.ds(eb, R_CHUNK)], elub_v)

        def rbody(j, _):
            ii = sbuf_v[pl.ds(j * 16, 16)]
            tse = plsc.load_gather(nts_v, [ii])
            rbuf_v[pl.ds(j * 16, 16)] = tse - elub_v[pl.ds(j * 16, 16)]
            return 0
        lax.fori_loop(0, R_CHUNK // 16, rbody, 0)
        pltpu.sync_copy(rbuf_v, relt_out.at[pl.ds(eb, R_CHUNK)])

    plsc.subcore_barrier()
    _dump_table(s1, o1, cid, sid)
    _dump_table(s2, o2, cid, sid)


@functools.partial(
    pl.kernel, mesh=_mesh, compiler_params=_cp,
    out_type=[jax.ShapeDtypeStruct((NC, NP, 144), jnp.float32)],
    scratch_types=[
        pltpu.VMEM_SHARED((NP, 144), jnp.float32),
        pltpu.VMEM((E_CHUNK,), jnp.int32),
        pltpu.VMEM((E_CHUNK,), jnp.int32),
        pltpu.VMEM((E_CHUNK, 256), jnp.float32),
        pltpu.VMEM((E_CHUNK, 128), jnp.float32),
        pltpu.VMEM((E_CHUNK, 128), jnp.float32),
        pltpu.VMEM((E_CHUNK, 144), jnp.float32),
        pltpu.SemaphoreType.DMA,
        pltpu.SemaphoreType.DMA,
        pltpu.SemaphoreType.DMA,
    ],
)
def _sc_d(esrc_hbm, edst_hbm, kv_hbm, q_hbm, etab_hbm,
          acc_out,
          acc, cidxs_v, cidxd_v, kvb_v, qb_v, eb_v, rowsb_v,
          sem1, sem2, sem3):
    cid = lax.axis_index("c")
    sid = lax.axis_index("s")
    wid = sid * NC + cid

    _zero2d(rowsb_v, E_CHUNK, 144)
    for j in range(632 // E_CHUNK):
        pltpu.sync_copy(rowsb_v, acc.at[pl.ds(sid * 632 + j * E_CHUNK, E_CHUNK)])
    pltpu.sync_copy(rowsb_v.at[pl.ds(0, 632 % E_CHUNK)],
                    acc.at[pl.ds(sid * 632 + (632 // E_CHUNK) * E_CHUNK,
                                 632 % E_CHUNK)])
    plsc.subcore_barrier()

    inv_sqrt = jnp.float32(128.0 ** -0.5)
    lanes = lax.iota(jnp.int32, 16)
    umask = jnp.where(lanes == 0, jnp.float32(1.0), jnp.float32(0.0))

    def chunk(chi, _):
        base = wid * E_PER_TILE + chi * E_CHUNK
        pltpu.sync_copy(esrc_hbm.at[pl.ds(base, E_CHUNK)], cidxs_v)
        pltpu.sync_copy(edst_hbm.at[pl.ds(base, E_CHUNK)], cidxd_v)
        cp1 = pltpu.async_copy(kv_hbm.at[cidxs_v], kvb_v, sem1)
        cp2 = pltpu.async_copy(q_hbm.at[cidxd_v], qb_v, sem2)
        cp3 = pltpu.async_copy(etab_hbm.at[pl.ds(base, E_CHUNK)], eb_v, sem3)
        cp1.wait()
        cp2.wait()
        cp3.wait()

        def edge(i, _):
            d16 = jnp.zeros((16,), jnp.float32)
            for c in range(8):
                qv = qb_v[i, pl.ds(c * 16, 16)]
                kvv = kvb_v[i, pl.ds(c * 16, 16)]
                ev = eb_v[i, pl.ds(c * 16, 16)]
                d16 = d16 + qv * (kvv + ev)
            s = jnp.sum(d16) * inv_sqrt
            svec = jnp.exp(jnp.full((16,), s, jnp.float32))
            for c in range(8):
                vv = kvb_v[i, pl.ds(128 + c * 16, 16)]
                ev = eb_v[i, pl.ds(c * 16, 16)]
                rowsb_v[i, pl.ds(c * 16, 16)] = svec * (vv + ev)
            rowsb_v[i, pl.ds(128, 16)] = svec * umask
            return 0
        lax.fori_loop(0, E_CHUNK, edge, 0)
        pltpu.sync_copy(rowsb_v, acc.at[cidxd_v], add=True)
        return 0
    lax.fori_loop(0, E_PER_TILE // E_CHUNK, chunk, 0)

    plsc.subcore_barrier()
    _dump_table(acc, acc_out, cid, sid)


def _tc0_body(ts_ref, wt_ref, bt_ref, out_ref):
    out_ref[...] = jnp.cos(ts_ref[...] * wt_ref[...] + bt_ref[...])


def _tcb_node_body(ms0, ms1, md0, md1, mt0, mt1, me0, me1, mty0, mty1,
                   mem_ref, x_ref, w1, w2, w3, w4, w6, bih, whh, bhh,
                   wq, wk, wv, wsk,
                   upd_ref, q_ref, kv_ref, skip_ref):
    f32 = jnp.float32
    ms = ms0[...] + ms1[...]
    md = md0[...] + md1[...]
    mt = mt0[...] + mt1[...]
    me = me0[...] + me1[...]
    mty = mty0[...] + mty1[...]
    cnt = mty[:, 8:9]
    inv = 1.0 / jnp.maximum(cnt, 1.0)
    gi = (jnp.dot(ms * inv, w1[...], preferred_element_type=f32)
          + jnp.dot(md * inv, w2[...], preferred_element_type=f32)
          + jnp.dot(mt * inv, w3[...], preferred_element_type=f32)
          + jnp.dot(me * inv, w4[...], preferred_element_type=f32)
          + jnp.dot(mty * inv, w6[...], preferred_element_type=f32)
          + bih[...])
    mem = mem_ref[...]
    gh = jnp.dot(mem, whh[...], preferred_element_type=f32) + bhh[...]
    d = 64
    r = jax.nn.sigmoid(gi[:, :d] + gh[:, :d])
    z = jax.nn.sigmoid(gi[:, d:2 * d] + gh[:, d:2 * d])
    n_ = jnp.tanh(gi[:, 2 * d:] + r * gh[:, 2 * d:])
    new_mem = (1.0 - z) * n_ + z * mem
    upd = jnp.where(cnt > 0, new_mem, mem)
    upd_ref[...] = upd
    x_cat = jnp.concatenate([x_ref[...], upd], axis=1)
    q_ref[...] = jnp.dot(x_cat, wq[...], preferred_element_type=f32)
    k = jnp.dot(x_cat, wk[...], preferred_element_type=f32)
    v = jnp.dot(x_cat, wv[...], preferred_element_type=f32)
    kv_ref[...] = jnp.concatenate([k, v], axis=1)
    skip_ref[...] = jnp.dot(x_cat, wsk[...], preferred_element_type=f32)


def _tcb_edge_body(rel_ref, attr_ref, wt_ref, bt_ref, wet_ref, wea_ref, e_ref):
    cosv = jnp.cos(rel_ref[...] * wt_ref[...] + bt_ref[...])
    e_ref[...] = (jnp.dot(cosv, wet_ref[...], preferred_element_type=jnp.float32)
                  + jnp.dot(attr_ref[...], wea_ref[...],
                            preferred_element_type=jnp.float32))


def _tcd_body(a0_ref, a1_ref, skip_ref, wlin_ref, blin_ref, out_ref):
    acc = a0_ref[...] + a1_ref[...]
    denom = jnp.maximum(acc[:, 128:129], 1e-16)
    out = acc[:, :128] / denom + skip_ref[...]
    out_ref[...] = jnp.dot(out, wlin_ref[...],
                           preferred_element_type=jnp.float32) + blin_ref[...]


def kernel(event_type_ids, event_src_ids, event_dst_ids, event_embeddings,
           event_timestamps, x, edge_index, edge_attr, edge_last_update,
           batch, memory, emb_table, Wt, bt, W_ih, b_ih, W_hh, b_hh,
           Wq, Wk, Wv, We, Wskip, Wlin, blin):
    f32 = jnp.float32
    srcs = edge_index[0]
    dsts = edge_index[1]
    embt16 = jnp.concatenate([emb_table, jnp.ones((7, 1), f32),
                              jnp.zeros((7, 7), f32)], axis=1).reshape(112)
    W1 = W_ih[8:72]
    W2 = W_ih[72:136]
    W3 = W_ih[136:168]
    W4 = W_ih[168:296]
    W6 = jnp.concatenate([W_ih[0:8], jnp.zeros((8, 192), f32)], axis=0)
    WeT = We[:32]
    WeA = We[32:]

    time_embs = pl.pallas_call(
        _tc0_body,
        out_shape=jax.ShapeDtypeStruct((B, 32), f32),
    )(event_timestamps.reshape(B, 1), Wt, bt.reshape(1, 32))

    a1, a2 = _sc_a(event_src_ids, event_dst_ids, memory)
    (bb,) = _sc_b(event_src_ids, event_dst_ids, event_embeddings)
    c1, c2, rel_t = _sc_c(event_src_ids, event_dst_ids, event_type_ids,
                          time_embs, batch, event_timestamps, srcs,
                          edge_last_update, embt16)

    nb = 1000
    grid_n = N // nb
    row = lambda i: (i, 0)
    full = lambda i: (0, 0)
    upd, q, kv, skip = pl.pallas_call(
        _tcb_node_body,
        grid=(grid_n,),
        in_specs=[
            pl.BlockSpec((nb, 64), row), pl.BlockSpec((nb, 64), row),
            pl.BlockSpec((nb, 64), row), pl.BlockSpec((nb, 64), row),
            pl.BlockSpec((nb, 32), row), pl.BlockSpec((nb, 32), row),
            pl.BlockSpec((nb, 128), row), pl.BlockSpec((nb, 128), row),
            pl.BlockSpec((nb, 16), row), pl.BlockSpec((nb, 16), row),
            pl.BlockSpec((nb, 64), row), pl.BlockSpec((nb, 128), row),
            pl.BlockSpec((64, 192), full), pl.BlockSpec((64, 192), full),
            pl.BlockSpec((32, 192), full), pl.BlockSpec((128, 192), full),
            pl.BlockSpec((16, 192), full),
            pl.BlockSpec((1, 192), full), pl.BlockSpec((64, 192), full),
            pl.BlockSpec((1, 192), full),
            pl.BlockSpec((192, 128), full), pl.BlockSpec((192, 128), full),
            pl.BlockSpec((192, 128), full), pl.BlockSpec((192, 128), full),
        ],
        out_specs=[
            pl.BlockSpec((nb, 64), row), pl.BlockSpec((nb, 128), row),
            pl.BlockSpec((nb, 256), row), pl.BlockSpec((nb, 128), row),
        ],
        out_shape=[
            jax.ShapeDtypeStruct((N, 64), f32),
            jax.ShapeDtypeStruct((N, 128), f32),
            jax.ShapeDtypeStruct((N, 256), f32),
            jax.ShapeDtypeStruct((N, 128), f32),
        ],
    )(a1[0, :N], a1[1, :N], a2[0, :N], a2[1, :N], c1[0, :N], c1[1, :N],
      bb[0, :N], bb[1, :N], c2[0, :N], c2[1, :N], memory, x,
      W1, W2, W3, W4, W6, b_ih.reshape(1, 192), W_hh, b_hh.reshape(1, 192),
      Wq, Wk, Wv, Wskip)

    ebk = 4000
    e_tab = pl.pallas_call(
        _tcb_edge_body,
        grid=(E // ebk,),
        in_specs=[
            pl.BlockSpec((ebk, 1), row), pl.BlockSpec((ebk, 128), row),
            pl.BlockSpec((1, 32), full), pl.BlockSpec((1, 32), full),
            pl.BlockSpec((32, 128), full), pl.BlockSpec((128, 128), full),
        ],
        out_specs=pl.BlockSpec((ebk, 128), row),
        out_shape=jax.ShapeDtypeStruct((E, 128), f32),
    )(rel_t.reshape(E, 1), edge_attr, Wt, bt.reshape(1, 32), WeT, WeA)

    (acc_parts,) = _sc_d(srcs, dsts, kv, q, e_tab)

    node_embeddings = pl.pallas_call(
        _tcd_body,
        grid=(grid_n,),
        in_specs=[
            pl.BlockSpec((nb, 144), row), pl.BlockSpec((nb, 144), row),
            pl.BlockSpec((nb, 128), row),
            pl.BlockSpec((128, 128), full), pl.BlockSpec((1, 128), full),
        ],
        out_specs=pl.BlockSpec((nb, 128), row),
        out_shape=jax.ShapeDtypeStruct((N, 128), f32),
    )(acc_parts[0, :N], acc_parts[1, :N], skip, Wlin, blin.reshape(1, 128))

    return node_embeddings, upd


# speedup vs baseline: 4.9544x; 1.1359x over previous
import functools

import jax
import jax.numpy as jnp
from jax import lax
from jax.experimental import pallas as pl
from jax.experimental.pallas import tpu as pltpu
from jax.experimental.pallas import tpu_sc as plsc

N = 10000
NP = 10112
E = 320000
B = 4096
NC = 2
NS = 16
NW = NC * NS

EV_CHUNK = 128
E_PER_TILE = E // NW
E_CHUNK = 32
E_TAIL = E_PER_TILE % E_CHUNK
R_CHUNK = 2000
SLABS = (128, 128, 128, 128, 120)

_mesh = plsc.VectorSubcoreMesh(core_axis_name="c", subcore_axis_name="s")
_cp = pltpu.CompilerParams(needs_layout_passes=False, use_tc_tiling_on_sc=False)


def _zero2d(ref, rows, cols):
    def body(i, _):
        for c in range(cols // 16):
            ref[i, pl.ds(c * 16, 16)] = jnp.zeros((16,), jnp.float32)
        return 0
    lax.fori_loop(0, rows, body, 0)


def _fill_table(zbuf, table, sid):
    for j, zr in enumerate(SLABS):
        pltpu.sync_copy(zbuf.at[pl.ds(0, zr)],
                        table.at[pl.ds(sid * 632 + j * 128, zr)])


def _dump_table(table, out, cid, sid):
    for j, zr in enumerate(SLABS):
        r0 = sid * 632 + j * 128
        pltpu.sync_copy(table.at[pl.ds(r0, zr)], out.at[cid, pl.ds(r0, zr)])


@functools.partial(
    pl.kernel, mesh=_mesh, compiler_params=_cp,
    out_type=[
        jax.ShapeDtypeStruct((NC, NP, 64), jnp.float32),
        jax.ShapeDtypeStruct((NC, NP, 64), jnp.float32),
    ],
    scratch_types=[
        pltpu.VMEM_SHARED((NP, 64), jnp.float32),
        pltpu.VMEM_SHARED((NP, 64), jnp.float32),
        pltpu.VMEM((EV_CHUNK, 64), jnp.float32),
        pltpu.VMEM((EV_CHUNK, 64), jnp.float32),
        pltpu.VMEM((EV_CHUNK,), jnp.int32),
        pltpu.VMEM((EV_CHUNK,), jnp.int32),
        pltpu.SemaphoreType.DMA,
        pltpu.SemaphoreType.DMA,
    ],
)
def _sc_a(src_ids, dst_ids, mem_hbm, o1, o2,
          s1, s2, smem_v, dmem_v, idxs_v, idxd_v, sem1, sem2):
    cid = lax.axis_index("c")
    sid = lax.axis_index("s")
    _zero2d(smem_v, EV_CHUNK, 64)
    _fill_table(smem_v, s1, sid)
    _fill_table(smem_v, s2, sid)
    plsc.subcore_barrier()

    ev0 = cid * (B // NC) + sid * EV_CHUNK
    pltpu.sync_copy(src_ids.at[pl.ds(ev0, EV_CHUNK)], idxs_v)
    pltpu.sync_copy(dst_ids.at[pl.ds(ev0, EV_CHUNK)], idxd_v)
    cp1 = pltpu.async_copy(mem_hbm.at[idxs_v], smem_v, sem1)
    cp2 = pltpu.async_copy(mem_hbm.at[idxd_v], dmem_v, sem2)
    cp1.wait()
    cp2.wait()
    pltpu.sync_copy(smem_v, s1.at[idxs_v], add=True)
    pltpu.sync_copy(smem_v, s1.at[idxd_v], add=True)
    pltpu.sync_copy(dmem_v, s2.at[idxs_v], add=True)
    pltpu.sync_copy(dmem_v, s2.at[idxd_v], add=True)

    plsc.subcore_barrier()
    _dump_table(s1, o1, cid, sid)
    _dump_table(s2, o2, cid, sid)


@functools.partial(
    pl.kernel, mesh=_mesh, compiler_params=_cp,
    out_type=[jax.ShapeDtypeStruct((NC, NP, 128), jnp.float32)],
    scratch_types=[
        pltpu.VMEM_SHARED((NP, 128), jnp.float32),
        pltpu.VMEM((EV_CHUNK, 128), jnp.float32),
        pltpu.VMEM((EV_CHUNK,), jnp.int32),
        pltpu.VMEM((EV_CHUNK,), jnp.int32),
    ],
)
def _sc_b(src_ids, dst_ids, eemb_hbm, o1, s1, ebuf_v, idxs_v, idxd_v):
    cid = lax.axis_index("c")
    sid = lax.axis_index("s")
    _zero2d(ebuf_v, EV_CHUNK, 128)
    _fill_table(ebuf_v, s1, sid)
    plsc.subcore_barrier()

    ev0 = cid * (B // NC) + sid * EV_CHUNK
    pltpu.sync_copy(src_ids.at[pl.ds(ev0, EV_CHUNK)], idxs_v)
    pltpu.sync_copy(dst_ids.at[pl.ds(ev0, EV_CHUNK)], idxd_v)
    pltpu.sync_copy(eemb_hbm.at[pl.ds(ev0, EV_CHUNK)], ebuf_v)
    pltpu.sync_copy(ebuf_v, s1.at[idxs_v], add=True)
    pltpu.sync_copy(ebuf_v, s1.at[idxd_v], add=True)

    plsc.subcore_barrier()
    _dump_table(s1, o1, cid, sid)


@functools.partial(
    pl.kernel, mesh=_mesh, compiler_params=_cp,
    out_type=[
        jax.ShapeDtypeStruct((NC, NP, 32), jnp.float32),
        jax.ShapeDtypeStruct((NC, NP, 16), jnp.float32),
        jax.ShapeDtypeStruct((E,), jnp.float32),
    ],
    scratch_types=[
        pltpu.VMEM_SHARED((NP, 32), jnp.float32),
        pltpu.VMEM_SHARED((NP, 16), jnp.float32),
        pltpu.VMEM((EV_CHUNK, 32), jnp.float32),
        pltpu.VMEM((EV_CHUNK, 16), jnp.float32),
        pltpu.VMEM((EV_CHUNK,), jnp.int32),
        pltpu.VMEM((EV_CHUNK,), jnp.int32),
        pltpu.VMEM((EV_CHUNK,), jnp.int32),
        pltpu.VMEM((112,), jnp.float32),
        pltpu.VMEM((B,), jnp.float32),
        pltpu.VMEM((R_CHUNK,), jnp.int32),
        pltpu.VMEM((N,), jnp.float32),
        pltpu.VMEM((R_CHUNK,), jnp.int32),
        pltpu.VMEM((R_CHUNK,), jnp.float32),
        pltpu.VMEM((R_CHUNK,), jnp.float32),
    ],
)
def _sc_c(src_ids, dst_ids, type_ids, time_embs, batch_hbm, ts_hbm,
          esrc_hbm, elu_hbm, embt_hbm,
          o1, o2, relt_out,
          s1, s2, tbuf_v, tybuf_v, idxs_v, idxd_v, tids_v, embt_v,
          tsv, bch_v, nts_v, sbuf_v, elub_v, rbuf_v):
    cid = lax.axis_index("c")
    sid = lax.axis_index("s")
    wid = sid * NC + cid
    _zero2d(tbuf_v, EV_CHUNK, 32)
    _fill_table(tbuf_v, s1, sid)
    _zero2d(tybuf_v, EV_CHUNK, 16)
    _fill_table(tybuf_v, s2, sid)
    plsc.subcore_barrier()

    pltpu.sync_copy(embt_hbm, embt_v)
    ev0 = cid * (B // NC) + sid * EV_CHUNK
    pltpu.sync_copy(src_ids.at[pl.ds(ev0, EV_CHUNK)], idxs_v)
    pltpu.sync_copy(dst_ids.at[pl.ds(ev0, EV_CHUNK)], idxd_v)
    pltpu.sync_copy(time_embs.at[pl.ds(ev0, EV_CHUNK)], tbuf_v)
    pltpu.sync_copy(type_ids.at[pl.ds(ev0, EV_CHUNK)], tids_v)
    lanes = lax.iota(jnp.int32, 16)

    def tyb(g, _):
        tv = tids_v[pl.ds(g * 16, 16)]
        for j in range(16):
            ti = tv[j]
            vals = plsc.load_gather(embt_v, [ti * 16 + lanes])
            tybuf_v[g * 16 + j, pl.ds(0, 16)] = vals
        return 0
    lax.fori_loop(0, EV_CHUNK // 16, tyb, 0)

    pltpu.sync_copy(tbuf_v, s1.at[idxs_v], add=True)
    pltpu.sync_copy(tbuf_v, s1.at[idxd_v], add=True)
    pltpu.sync_copy(tybuf_v, s2.at[idxs_v], add=True)
    pltpu.sync_copy(tybuf_v, s2.at[idxd_v], add=True)

    pltpu.sync_copy(ts_hbm, tsv)
    for m in range(N // R_CHUNK):
        pltpu.sync_copy(batch_hbm.at[pl.ds(m * R_CHUNK, R_CHUNK)], bch_v)

        def nbody(j, _):
            idx = bch_v[pl.ds(j * 16, 16)]
            nts_v[pl.ds(m * R_CHUNK + j * 16, 16)] = plsc.load_gather(tsv, [idx])
            return 0
        lax.fori_loop(0, R_CHUNK // 16, nbody, 0)

    for t in range(E_PER_TILE // R_CHUNK):
        eb = wid * E_PER_TILE + t * R_CHUNK
        pltpu.sync_copy(esrc_hbm.at[pl.ds(eb, R_CHUNK)], sbuf_v)
        pltpu.sync_copy(elu_hbm.at[pl.ds(eb, R_CHUNK)], elub_v)

        def rbody(j, _):
            ii = sbuf_v[pl.ds(j * 16, 16)]
            tse = plsc.load_gather(nts_v, [ii])
            rbuf_v[pl.ds(j * 16, 16)] = tse - elub_v[pl.ds(j * 16, 16)]
            return 0
        lax.fori_loop(0, R_CHUNK // 16, rbody, 0)
        pltpu.sync_copy(rbuf_v, relt_out.at[pl.ds(eb, R_CHUNK)])

    plsc.subcore_barrier()
    _dump_table(s1, o1, cid, sid)
    _dump_table(s2, o2, cid, sid)


@functools.partial(
    pl.kernel, mesh=_mesh, compiler_params=_cp,
    out_type=[jax.ShapeDtypeStruct((NC, NP, 144), jnp.float32)],
    scratch_types=[
        pltpu.VMEM_SHARED((NP, 144), jnp.float32),
        pltpu.VMEM((2, E_CHUNK), jnp.int32),
        pltpu.VMEM((2, E_CHUNK), jnp.int32),
        pltpu.VMEM((2, E_CHUNK, 256), jnp.float32),
        pltpu.VMEM((2, E_CHUNK, 128), jnp.float32),
        pltpu.VMEM((2, E_CHUNK, 128), jnp.float32),
        pltpu.VMEM((E_CHUNK, 144), jnp.float32),
        pltpu.VMEM((E_TAIL,), jnp.int32),
        pltpu.VMEM((E_TAIL,), jnp.int32),
        pltpu.SemaphoreType.DMA,
        pltpu.SemaphoreType.DMA,
    ],
)
def _sc_d(esrc_hbm, edst_hbm, kv_hbm, q_hbm, etab_hbm,
          acc_out,
          acc, cidxs_v, cidxd_v, kvb_v, qb_v, eb_v, rowsb_v,
          tidxs_v, tidxd_v, sem0, sem1):
    cid = lax.axis_index("c")
    sid = lax.axis_index("s")
    wid = sid * NC + cid
    sems = (sem0, sem1)

    _zero2d(rowsb_v, E_CHUNK, 144)
    nz = 632 // E_CHUNK
    for j in range(nz):
        pltpu.sync_copy(rowsb_v, acc.at[pl.ds(sid * 632 + j * E_CHUNK, E_CHUNK)])
    if 632 % E_CHUNK:
        pltpu.sync_copy(rowsb_v.at[pl.ds(0, 632 % E_CHUNK)],
                        acc.at[pl.ds(sid * 632 + nz * E_CHUNK, 632 % E_CHUNK)])
    plsc.subcore_barrier()

    inv_sqrt = jnp.float32(128.0 ** -0.5)
    lanes = lax.iota(jnp.int32, 16)
    umask = jnp.where(lanes == 0, jnp.float32(1.0), jnp.float32(0.0))
    e0 = wid * E_PER_TILE
    NCH = E_PER_TILE // E_CHUNK

    def start(slot, base):
        pltpu.sync_copy(esrc_hbm.at[pl.ds(base, E_CHUNK)], cidxs_v.at[slot])
        pltpu.sync_copy(edst_hbm.at[pl.ds(base, E_CHUNK)], cidxd_v.at[slot])
        pltpu.async_copy(kv_hbm.at[cidxs_v.at[slot]], kvb_v.at[slot], sems[slot])
        pltpu.async_copy(q_hbm.at[cidxd_v.at[slot]], qb_v.at[slot], sems[slot])
        pltpu.async_copy(etab_hbm.at[pl.ds(base, E_CHUNK)], eb_v.at[slot], sems[slot])

    def drain(slot):
        pltpu.make_async_copy(kv_hbm.at[cidxs_v.at[slot]], kvb_v.at[slot], sems[slot]).wait()
        pltpu.make_async_copy(q_hbm.at[cidxd_v.at[slot]], qb_v.at[slot], sems[slot]).wait()
        pltpu.make_async_copy(etab_hbm.at[pl.ds(0, E_CHUNK)], eb_v.at[slot], sems[slot]).wait()

    def compute(slot, nrows):
        def edge(i, _):
            d16 = jnp.zeros((16,), jnp.float32)
            for c in range(8):
                qv = qb_v[slot, i, pl.ds(c * 16, 16)]
                kvv = kvb_v[slot, i, pl.ds(c * 16, 16)]
                ev = eb_v[slot, i, pl.ds(c * 16, 16)]
                d16 = d16 + qv * (kvv + ev)
            s = jnp.sum(d16) * inv_sqrt
            svec = jnp.exp(jnp.full((16,), s, jnp.float32))
            for c in range(8):
                vv = kvb_v[slot, i, pl.ds(128 + c * 16, 16)]
                ev = eb_v[slot, i, pl.ds(c * 16, 16)]
                rowsb_v[i, pl.ds(c * 16, 16)] = svec * (vv + ev)
            rowsb_v[i, pl.ds(128, 16)] = svec * umask
            return 0
        lax.fori_loop(0, nrows, edge, 0)
        pltpu.sync_copy(rowsb_v, acc.at[cidxd_v.at[slot]], add=True)

    start(0, e0)

    def pair(j, _):
        start(1, e0 + (2 * j + 1) * E_CHUNK)
        drain(0)
        compute(0, E_CHUNK)

        @pl.when(2 * j + 2 < NCH)
        def _():
            start(0, e0 + (2 * j + 2) * E_CHUNK)
        drain(1)
        compute(1, E_CHUNK)
        return 0
    lax.fori_loop(0, NCH // 2, pair, 0)

    tbase = e0 + NCH * E_CHUNK
    pltpu.sync_copy(esrc_hbm.at[pl.ds(tbase, E_TAIL)], tidxs_v)
    pltpu.sync_copy(edst_hbm.at[pl.ds(tbase, E_TAIL)], tidxd_v)
    tc1 = pltpu.async_copy(kv_hbm.at[tidxs_v], kvb_v.at[0, pl.ds(0, E_TAIL)], sem0)
    tc2 = pltpu.async_copy(q_hbm.at[tidxd_v], qb_v.at[0, pl.ds(0, E_TAIL)], sem0)
    tc3 = pltpu.async_copy(etab_hbm.at[pl.ds(tbase, E_TAIL)],
                           eb_v.at[0, pl.ds(0, E_TAIL)], sem0)
    tc1.wait()
    tc2.wait()
    tc3.wait()

    def tedge(i, _):
        d16 = jnp.zeros((16,), jnp.float32)
        for c in range(8):
            d16 = d16 + qb_v[0, i, pl.ds(c * 16, 16)] * (
                kvb_v[0, i, pl.ds(c * 16, 16)] + eb_v[0, i, pl.ds(c * 16, 16)])
        s = jnp.sum(d16) * inv_sqrt
        svec = jnp.exp(jnp.full((16,), s, jnp.float32))
        for c in range(8):
            rowsb_v[i, pl.ds(c * 16, 16)] = svec * (
                kvb_v[0, i, pl.ds(128 + c * 16, 16)] + eb_v[0, i, pl.ds(c * 16, 16)])
        rowsb_v[i, pl.ds(128, 16)] = svec * umask
        return 0
    lax.fori_loop(0, E_TAIL, tedge, 0)
    pltpu.sync_copy(rowsb_v.at[pl.ds(0, E_TAIL)], acc.at[tidxd_v], add=True)

    plsc.subcore_barrier()
    _dump_table(acc, acc_out, cid, sid)


def _tc0_body(ts_ref, wt_ref, bt_ref, out_ref):
    out_ref[...] = jnp.cos(ts_ref[...] * wt_ref[...] + bt_ref[...])


def _tcb_node_body(ms0, ms1, md0, md1, mt0, mt1, me0, me1, mty0, mty1,
                   mem_ref, x_ref, w1, w2, w3, w4, w6, bih, whh, bhh,
                   wq, wk, wv, wsk,
                   upd_ref, q_ref, kv_ref, skip_ref):
    f32 = jnp.float32
    ms = ms0[...] + ms1[...]
    md = md0[...] + md1[...]
    mt = mt0[...] + mt1[...]
    me = me0[...] + me1[...]
    mty = mty0[...] + mty1[...]
    cnt = mty[:, 8:9]
    inv = 1.0 / jnp.maximum(cnt, 1.0)
    gi = (jnp.dot(ms * inv, w1[...], preferred_element_type=f32)
          + jnp.dot(md * inv, w2[...], preferred_element_type=f32)
          + jnp.dot(mt * inv, w3[...], preferred_element_type=f32)
          + jnp.dot(me * inv, w4[...], preferred_element_type=f32)
          + jnp.dot(mty * inv, w6[...], preferred_element_type=f32)
          + bih[...])
    mem = mem_ref[...]
    gh = jnp.dot(mem, whh[...], preferred_element_type=f32) + bhh[...]
    d = 64
    r = jax.nn.sigmoid(gi[:, :d] + gh[:, :d])
    z = jax.nn.sigmoid(gi[:, d:2 * d] + gh[:, d:2 * d])
    n_ = jnp.tanh(gi[:, 2 * d:] + r * gh[:, 2 * d:])
    new_mem = (1.0 - z) * n_ + z * mem
    upd = jnp.where(cnt > 0, new_mem, mem)
    upd_ref[...] = upd
    x_cat = jnp.concatenate([x_ref[...], upd], axis=1)
    q_ref[...] = jnp.dot(x_cat, wq[...], preferred_element_type=f32)
    k = jnp.dot(x_cat, wk[...], preferred_element_type=f32)
    v = jnp.dot(x_cat, wv[...], preferred_element_type=f32)
    kv_ref[...] = jnp.concatenate([k, v], axis=1)
    skip_ref[...] = jnp.dot(x_cat, wsk[...], preferred_element_type=f32)


def _tcb_edge_body(rel_ref, attr_ref, wt_ref, bt_ref, wet_ref, wea_ref, e_ref):
    cosv = jnp.cos(rel_ref[...] * wt_ref[...] + bt_ref[...])
    e_ref[...] = (jnp.dot(cosv, wet_ref[...], preferred_element_type=jnp.float32)
                  + jnp.dot(attr_ref[...], wea_ref[...],
                            preferred_element_type=jnp.float32))


def _tcd_body(a0_ref, a1_ref, skip_ref, wlin_ref, blin_ref, out_ref):
    acc = a0_ref[...] + a1_ref[...]
    denom = jnp.maximum(acc[:, 128:129], 1e-16)
    out = acc[:, :128] / denom + skip_ref[...]
    out_ref[...] = jnp.dot(out, wlin_ref[...],
                           preferred_element_type=jnp.float32) + blin_ref[...]


def kernel(event_type_ids, event_src_ids, event_dst_ids, event_embeddings,
           event_timestamps, x, edge_index, edge_attr, edge_last_update,
           batch, memory, emb_table, Wt, bt, W_ih, b_ih, W_hh, b_hh,
           Wq, Wk, Wv, We, Wskip, Wlin, blin):
    f32 = jnp.float32
    srcs = edge_index[0]
    dsts = edge_index[1]
    embt16 = jnp.concatenate([emb_table, jnp.ones((7, 1), f32),
                              jnp.zeros((7, 7), f32)], axis=1).reshape(112)
    W1 = W_ih[8:72]
    W2 = W_ih[72:136]
    W3 = W_ih[136:168]
    W4 = W_ih[168:296]
    W6 = jnp.concatenate([W_ih[0:8], jnp.zeros((8, 192), f32)], axis=0)
    WeT = We[:32]
    WeA = We[32:]

    time_embs = pl.pallas_call(
        _tc0_body,
        out_shape=jax.ShapeDtypeStruct((B, 32), f32),
    )(event_timestamps.reshape(B, 1), Wt, bt.reshape(1, 32))

    a1, a2 = _sc_a(event_src_ids, event_dst_ids, memory)
    (bb,) = _sc_b(event_src_ids, event_dst_ids, event_embeddings)
    c1, c2, rel_t = _sc_c(event_src_ids, event_dst_ids, event_type_ids,
                          time_embs, batch, event_timestamps, srcs,
                          edge_last_update, embt16)

    nb = 1000
    grid_n = N // nb
    row = lambda i: (i, 0)
    full = lambda i: (0, 0)
    upd, q, kv, skip = pl.pallas_call(
        _tcb_node_body,
        grid=(grid_n,),
        in_specs=[
            pl.BlockSpec((nb, 64), row), pl.BlockSpec((nb, 64), row),
            pl.BlockSpec((nb, 64), row), pl.BlockSpec((nb, 64), row),
            pl.BlockSpec((nb, 32), row), pl.BlockSpec((nb, 32), row),
            pl.BlockSpec((nb, 128), row), pl.BlockSpec((nb, 128), row),
            pl.BlockSpec((nb, 16), row), pl.BlockSpec((nb, 16), row),
            pl.BlockSpec((nb, 64), row), pl.BlockSpec((nb, 128), row),
            pl.BlockSpec((64, 192), full), pl.BlockSpec((64, 192), full),
            pl.BlockSpec((32, 192), full), pl.BlockSpec((128, 192), full),
            pl.BlockSpec((16, 192), full),
            pl.BlockSpec((1, 192), full), pl.BlockSpec((64, 192), full),
            pl.BlockSpec((1, 192), full),
            pl.BlockSpec((192, 128), full), pl.BlockSpec((192, 128), full),
            pl.BlockSpec((192, 128), full), pl.BlockSpec((192, 128), full),
        ],
        out_specs=[
            pl.BlockSpec((nb, 64), row), pl.BlockSpec((nb, 128), row),
            pl.BlockSpec((nb, 256), row), pl.BlockSpec((nb, 128), row),
        ],
        out_shape=[
            jax.ShapeDtypeStruct((N, 64), f32),
            jax.ShapeDtypeStruct((N, 128), f32),
            jax.ShapeDtypeStruct((N, 256), f32),
            jax.ShapeDtypeStruct((N, 128), f32),
        ],
    )(a1[0, :N], a1[1, :N], a2[0, :N], a2[1, :N], c1[0, :N], c1[1, :N],
      bb[0, :N], bb[1, :N], c2[0, :N], c2[1, :N], memory, x,
      W1, W2, W3, W4, W6, b_ih.reshape(1, 192), W_hh, b_hh.reshape(1, 192),
      Wq, Wk, Wv, Wskip)

    ebk = 4000
    e_tab = pl.pallas_call(
        _tcb_edge_body,
        grid=(E // ebk,),
        in_specs=[
            pl.BlockSpec((ebk, 1), row), pl.BlockSpec((ebk, 128), row),
            pl.BlockSpec((1, 32), full), pl.BlockSpec((1, 32), full),
            pl.BlockSpec((32, 128), full), pl.BlockSpec((128, 128), full),
        ],
        out_specs=pl.BlockSpec((ebk, 128), row),
        out_shape=jax.ShapeDtypeStruct((E, 128), f32),
    )(rel_t.reshape(E, 1), edge_attr, Wt, bt.reshape(1, 32), WeT, WeA)

    (acc_parts,) = _sc_d(srcs, dsts, kv, q, e_tab)

    node_embeddings = pl.pallas_call(
        _tcd_body,
        grid=(grid_n,),
        in_specs=[
            pl.BlockSpec((nb, 144), row), pl.BlockSpec((nb, 144), row),
            pl.BlockSpec((nb, 128), row),
            pl.BlockSpec((128, 128), full), pl.BlockSpec((1, 128), full),
        ],
        out_specs=pl.BlockSpec((nb, 128), row),
        out_shape=jax.ShapeDtypeStruct((N, 128), f32),
    )(acc_parts[0, :N], acc_parts[1, :N], skip, Wlin, blin.reshape(1, 128))

    return node_embeddings, upd


# speedup vs baseline: 5.4964x; 1.1094x over previous
import functools

import jax
import jax.numpy as jnp
from jax import lax
from jax.experimental import pallas as pl
from jax.experimental.pallas import tpu as pltpu
from jax.experimental.pallas import tpu_sc as plsc

N = 10000
NP = 10112
E = 320000
B = 4096
NC = 2
NS = 16
NW = NC * NS

EV_CHUNK = 128
E_PER_TILE = E // NW
E_CHUNK = 32
E_TAIL = E_PER_TILE % E_CHUNK
R_CHUNK = 2000
SLABS = (128, 128, 128, 128, 120)

_mesh = plsc.VectorSubcoreMesh(core_axis_name="c", subcore_axis_name="s")
_cp = pltpu.CompilerParams(needs_layout_passes=False, use_tc_tiling_on_sc=False)


def _zero2d(ref, rows, cols):
    def body(i, _):
        for c in range(cols // 16):
            ref[i, pl.ds(c * 16, 16)] = jnp.zeros((16,), jnp.float32)
        return 0
    lax.fori_loop(0, rows, body, 0)


def _fill_table(zbuf, table, sid):
    for j, zr in enumerate(SLABS):
        pltpu.sync_copy(zbuf.at[pl.ds(0, zr)],
                        table.at[pl.ds(sid * 632 + j * 128, zr)])


def _dump_table(table, out, cid, sid):
    for j, zr in enumerate(SLABS):
        r0 = sid * 632 + j * 128
        pltpu.sync_copy(table.at[pl.ds(r0, zr)], out.at[cid, pl.ds(r0, zr)])


@functools.partial(
    pl.kernel, mesh=_mesh, compiler_params=_cp,
    out_type=[
        jax.ShapeDtypeStruct((NC, NP, 64), jnp.float32),
        jax.ShapeDtypeStruct((NC, NP, 64), jnp.float32),
    ],
    scratch_types=[
        pltpu.VMEM_SHARED((NP, 64), jnp.float32),
        pltpu.VMEM_SHARED((NP, 64), jnp.float32),
        pltpu.VMEM((EV_CHUNK, 64), jnp.float32),
        pltpu.VMEM((EV_CHUNK, 64), jnp.float32),
        pltpu.VMEM((EV_CHUNK,), jnp.int32),
        pltpu.VMEM((EV_CHUNK,), jnp.int32),
        pltpu.SemaphoreType.DMA,
        pltpu.SemaphoreType.DMA,
    ],
)
def _sc_a(src_ids, dst_ids, mem_hbm, o1, o2,
          s1, s2, smem_v, dmem_v, idxs_v, idxd_v, sem1, sem2):
    cid = lax.axis_index("c")
    sid = lax.axis_index("s")
    _zero2d(smem_v, EV_CHUNK, 64)
    _fill_table(smem_v, s1, sid)
    _fill_table(smem_v, s2, sid)
    plsc.subcore_barrier()

    ev0 = cid * (B // NC) + sid * EV_CHUNK
    pltpu.sync_copy(src_ids.at[pl.ds(ev0, EV_CHUNK)], idxs_v)
    pltpu.sync_copy(dst_ids.at[pl.ds(ev0, EV_CHUNK)], idxd_v)
    cp1 = pltpu.async_copy(mem_hbm.at[idxs_v], smem_v, sem1)
    cp2 = pltpu.async_copy(mem_hbm.at[idxd_v], dmem_v, sem2)
    cp1.wait()
    cp2.wait()
    pltpu.sync_copy(smem_v, s1.at[idxs_v], add=True)
    pltpu.sync_copy(smem_v, s1.at[idxd_v], add=True)
    pltpu.sync_copy(dmem_v, s2.at[idxs_v], add=True)
    pltpu.sync_copy(dmem_v, s2.at[idxd_v], add=True)

    plsc.subcore_barrier()
    _dump_table(s1, o1, cid, sid)
    _dump_table(s2, o2, cid, sid)


@functools.partial(
    pl.kernel, mesh=_mesh, compiler_params=_cp,
    out_type=[jax.ShapeDtypeStruct((NC, NP, 128), jnp.float32)],
    scratch_types=[
        pltpu.VMEM_SHARED((NP, 128), jnp.float32),
        pltpu.VMEM((EV_CHUNK, 128), jnp.float32),
        pltpu.VMEM((EV_CHUNK,), jnp.int32),
        pltpu.VMEM((EV_CHUNK,), jnp.int32),
    ],
)
def _sc_b(src_ids, dst_ids, eemb_hbm, o1, s1, ebuf_v, idxs_v, idxd_v):
    cid = lax.axis_index("c")
    sid = lax.axis_index("s")
    _zero2d(ebuf_v, EV_CHUNK, 128)
    _fill_table(ebuf_v, s1, sid)
    plsc.subcore_barrier()

    ev0 = cid * (B // NC) + sid * EV_CHUNK
    pltpu.sync_copy(src_ids.at[pl.ds(ev0, EV_CHUNK)], idxs_v)
    pltpu.sync_copy(dst_ids.at[pl.ds(ev0, EV_CHUNK)], idxd_v)
    pltpu.sync_copy(eemb_hbm.at[pl.ds(ev0, EV_CHUNK)], ebuf_v)
    pltpu.sync_copy(ebuf_v, s1.at[idxs_v], add=True)
    pltpu.sync_copy(ebuf_v, s1.at[idxd_v], add=True)

    plsc.subcore_barrier()
    _dump_table(s1, o1, cid, sid)


@functools.partial(
    pl.kernel, mesh=_mesh, compiler_params=_cp,
    out_type=[
        jax.ShapeDtypeStruct((NC, NP, 32), jnp.float32),
        jax.ShapeDtypeStruct((NC, NP, 16), jnp.float32),
        jax.ShapeDtypeStruct((E,), jnp.float32),
    ],
    scratch_types=[
        pltpu.VMEM_SHARED((NP, 32), jnp.float32),
        pltpu.VMEM_SHARED((NP, 16), jnp.float32),
        pltpu.VMEM((EV_CHUNK, 32), jnp.float32),
        pltpu.VMEM((EV_CHUNK, 16), jnp.float32),
        pltpu.VMEM((EV_CHUNK,), jnp.int32),
        pltpu.VMEM((EV_CHUNK,), jnp.int32),
        pltpu.VMEM((EV_CHUNK,), jnp.int32),
        pltpu.VMEM((112,), jnp.float32),
        pltpu.VMEM((B,), jnp.float32),
        pltpu.VMEM((R_CHUNK,), jnp.int32),
        pltpu.VMEM((N,), jnp.float32),
        pltpu.VMEM((R_CHUNK,), jnp.int32),
        pltpu.VMEM((R_CHUNK,), jnp.float32),
        pltpu.VMEM((R_CHUNK,), jnp.float32),
    ],
)
def _sc_c(src_ids, dst_ids, type_ids, time_embs, batch_hbm, ts_hbm,
          esrc_hbm, elu_hbm, embt_hbm,
          o1, o2, relt_out,
          s1, s2, tbuf_v, tybuf_v, idxs_v, idxd_v, tids_v, embt_v,
          tsv, bch_v, nts_v, sbuf_v, elub_v, rbuf_v):
    cid = lax.axis_index("c")
    sid = lax.axis_index("s")
    wid = sid * NC + cid
    _zero2d(tbuf_v, EV_CHUNK, 32)
    _fill_table(tbuf_v, s1, sid)
    _zero2d(tybuf_v, EV_CHUNK, 16)
    _fill_table(tybuf_v, s2, sid)
    plsc.subcore_barrier()

    pltpu.sync_copy(embt_hbm, embt_v)
    ev0 = cid * (B // NC) + sid * EV_CHUNK
    pltpu.sync_copy(src_ids.at[pl.ds(ev0, EV_CHUNK)], idxs_v)
    pltpu.sync_copy(dst_ids.at[pl.ds(ev0, EV_CHUNK)], idxd_v)
    pltpu.sync_copy(time_embs.at[pl.ds(ev0, EV_CHUNK)], tbuf_v)
    pltpu.sync_copy(type_ids.at[pl.ds(ev0, EV_CHUNK)], tids_v)
    lanes = lax.iota(jnp.int32, 16)

    def tyb(g, _):
        tv = tids_v[pl.ds(g * 16, 16)]
        for j in range(16):
            ti = tv[j]
            vals = plsc.load_gather(embt_v, [ti * 16 + lanes])
            tybuf_v[g * 16 + j, pl.ds(0, 16)] = vals
        return 0
    lax.fori_loop(0, EV_CHUNK // 16, tyb, 0)

    pltpu.sync_copy(tbuf_v, s1.at[idxs_v], add=True)
    pltpu.sync_copy(tbuf_v, s1.at[idxd_v], add=True)
    pltpu.sync_copy(tybuf_v, s2.at[idxs_v], add=True)
    pltpu.sync_copy(tybuf_v, s2.at[idxd_v], add=True)

    pltpu.sync_copy(ts_hbm, tsv)
    for m in range(N // R_CHUNK):
        pltpu.sync_copy(batch_hbm.at[pl.ds(m * R_CHUNK, R_CHUNK)], bch_v)

        def nbody(j, _):
            idx = bch_v[pl.ds(j * 16, 16)]
            nts_v[pl.ds(m * R_CHUNK + j * 16, 16)] = plsc.load_gather(tsv, [idx])
            return 0
        lax.fori_loop(0, R_CHUNK // 16, nbody, 0)

    for t in range(E_PER_TILE // R_CHUNK):
        eb = wid * E_PER_TILE + t * R_CHUNK
        pltpu.sync_copy(esrc_hbm.at[pl.ds(eb, R_CHUNK)], sbuf_v)
        pltpu.sync_copy(elu_hbm.at[pl.ds(eb, R_CHUNK)], elub_v)

        def rbody(j, _):
            ii = sbuf_v[pl.ds(j * 16, 16)]
            tse = plsc.load_gather(nts_v, [ii])
            rbuf_v[pl.ds(j * 16, 16)] = tse - elub_v[pl.ds(j * 16, 16)]
            return 0
        lax.fori_loop(0, R_CHUNK // 16, rbody, 0)
        pltpu.sync_copy(rbuf_v, relt_out.at[pl.ds(eb, R_CHUNK)])

    plsc.subcore_barrier()
    _dump_table(s1, o1, cid, sid)
    _dump_table(s2, o2, cid, sid)


E_BLK = 26
N_BLK = 12


@functools.partial(
    pl.kernel, mesh=_mesh, compiler_params=_cp,
    out_type=[jax.ShapeDtypeStruct((NC, NP, 144), jnp.float32)],
    scratch_types=[
        pltpu.VMEM_SHARED((NP, 144), jnp.float32),
        pltpu.VMEM((E_BLK * E_CHUNK,), jnp.int32),
        pltpu.VMEM((E_BLK, E_CHUNK), jnp.int32),
        pltpu.VMEM((2, E_CHUNK, 256), jnp.float32),
        pltpu.VMEM((2, E_CHUNK, 128), jnp.float32),
        pltpu.VMEM((2, E_CHUNK, 128), jnp.float32),
        pltpu.VMEM((E_CHUNK, 144), jnp.float32),
        pltpu.VMEM((E_CHUNK,), jnp.int32),
        pltpu.VMEM((1, E_CHUNK), jnp.int32),
        pltpu.SemaphoreType.DMA,
        pltpu.SemaphoreType.DMA,
    ],
)
def _sc_d(esrc_hbm, edst2_hbm, kv_hbm, q_hbm, etab_hbm,
          acc_out,
          acc, sblk_v, dblk_v, kvb_v, qb_v, eb_v, rowsb_v,
          tidxs_v, tdx_v, sem0, sem1):
    cid = lax.axis_index("c")
    sid = lax.axis_index("s")
    wid = sid * NC + cid
    sems = (sem0, sem1)

    _zero2d(rowsb_v, E_CHUNK, 144)
    nz = 632 // E_CHUNK
    for j in range(nz):
        pltpu.sync_copy(rowsb_v, acc.at[pl.ds(sid * 632 + j * E_CHUNK, E_CHUNK)])
    if 632 % E_CHUNK:
        pltpu.sync_copy(rowsb_v.at[pl.ds(0, 632 % E_CHUNK)],
                        acc.at[pl.ds(sid * 632 + nz * E_CHUNK, 632 % E_CHUNK)])
    plsc.subcore_barrier()

    inv_sqrt = jnp.float32(128.0 ** -0.5)
    lanes = lax.iota(jnp.int32, 16)
    umask = jnp.where(lanes == 0, jnp.float32(1.0), jnp.float32(0.0))

    def start(slot, brow, k):
        pltpu.async_copy(kv_hbm.at[sblk_v.at[pl.ds(k * E_CHUNK, E_CHUNK)]],
                         kvb_v.at[slot], sems[slot])
        pltpu.async_copy(q_hbm.at[dblk_v.at[k]], qb_v.at[slot], sems[slot])
        pltpu.async_copy(etab_hbm.at[pl.ds((brow + k) * E_CHUNK, E_CHUNK)],
                         eb_v.at[slot], sems[slot])

    def drain(slot, brow, k):
        pltpu.make_async_copy(kv_hbm.at[sblk_v.at[pl.ds(k * E_CHUNK, E_CHUNK)]],
                              kvb_v.at[slot], sems[slot]).wait()
        pltpu.make_async_copy(q_hbm.at[dblk_v.at[k]], qb_v.at[slot], sems[slot]).wait()
        pltpu.make_async_copy(etab_hbm.at[pl.ds((brow + k) * E_CHUNK, E_CHUNK)],
                              eb_v.at[slot], sems[slot]).wait()

    def compute(slot, idx_ref):
        def edge(i, _):
            d16 = jnp.zeros((16,), jnp.float32)
            for c in range(8):
                qv = qb_v[slot, i, pl.ds(c * 16, 16)]
                kvv = kvb_v[slot, i, pl.ds(c * 16, 16)]
                ev = eb_v[slot, i, pl.ds(c * 16, 16)]
                d16 = d16 + qv * (kvv + ev)
            s = jnp.sum(d16) * inv_sqrt
            svec = jnp.exp(jnp.full((16,), s, jnp.float32))
            for c in range(8):
                vv = kvb_v[slot, i, pl.ds(128 + c * 16, 16)]
                ev = eb_v[slot, i, pl.ds(c * 16, 16)]
                rowsb_v[i, pl.ds(c * 16, 16)] = svec * (vv + ev)
            rowsb_v[i, pl.ds(128, 16)] = svec * umask
            return 0
        lax.fori_loop(0, E_CHUNK, edge, 0)
        pltpu.sync_copy(rowsb_v, acc.at[idx_ref], add=True)

    def block(blk, _):
        brow = (wid * N_BLK + blk) * E_BLK
        pltpu.sync_copy(esrc_hbm.at[pl.ds(brow * E_CHUNK, E_BLK * E_CHUNK)], sblk_v)
        pltpu.sync_copy(edst2_hbm.at[pl.ds(brow, E_BLK)], dblk_v)
        start(0, brow, 0)

        def pair(j, _):
            start(1, brow, 2 * j + 1)
            drain(0, brow, 2 * j)
            compute(0, dblk_v.at[2 * j])

            @pl.when(2 * j + 2 < E_BLK)
            def _():
                start(0, brow, 2 * j + 2)
            drain(1, brow, 2 * j + 1)
            compute(1, dblk_v.at[2 * j + 1])
            return 0
        lax.fori_loop(0, E_BLK // 2, pair, 0)
        return 0
    lax.fori_loop(0, N_BLK, block, 0)

    @pl.when(wid < 16)
    def _():
        trow = NW * N_BLK * E_BLK + wid
        pltpu.sync_copy(esrc_hbm.at[pl.ds(trow * E_CHUNK, E_CHUNK)], tidxs_v)
        pltpu.sync_copy(edst2_hbm.at[pl.ds(trow, 1)], tdx_v)
        tc1 = pltpu.async_copy(kv_hbm.at[tidxs_v], kvb_v.at[0], sem0)
        tc2 = pltpu.async_copy(q_hbm.at[tdx_v.at[0]], qb_v.at[0], sem0)
        tc3 = pltpu.async_copy(etab_hbm.at[pl.ds(trow * E_CHUNK, E_CHUNK)],
                               eb_v.at[0], sem0)
        tc1.wait()
        tc2.wait()
        tc3.wait()
        compute(0, tdx_v.at[0])

    plsc.subcore_barrier()
    _dump_table(acc, acc_out, cid, sid)


def _tc0_body(ts_ref, wt_ref, bt_ref, out_ref):
    out_ref[...] = jnp.cos(ts_ref[...] * wt_ref[...] + bt_ref[...])


def _tcb_node_body(ms0, ms1, md0, md1, mt0, mt1, me0, me1, mty0, mty1,
                   mem_ref, x_ref, w1, w2, w3, w4, w6, bih, whh, bhh,
                   wq, wk, wv, wsk,
                   upd_ref, q_ref, kv_ref, skip_ref):
    f32 = jnp.float32
    ms = ms0[...] + ms1[...]
    md = md0[...] + md1[...]
    mt = mt0[...] + mt1[...]
    me = me0[...] + me1[...]
    mty = mty0[...] + mty1[...]
    cnt = mty[:, 8:9]
    inv = 1.0 / jnp.maximum(cnt, 1.0)
    gi = (jnp.dot(ms * inv, w1[...], preferred_element_type=f32)
          + jnp.dot(md * inv, w2[...], preferred_element_type=f32)
          + jnp.dot(mt * inv, w3[...], preferred_element_type=f32)
          + jnp.dot(me * inv, w4[...], preferred_element_type=f32)
          + jnp.dot(mty * inv, w6[...], preferred_element_type=f32)
          + bih[...])
    mem = mem_ref[...]
    gh = jnp.dot(mem, whh[...], preferred_element_type=f32) + bhh[...]
    d = 64
    r = jax.nn.sigmoid(gi[:, :d] + gh[:, :d])
    z = jax.nn.sigmoid(gi[:, d:2 * d] + gh[:, d:2 * d])
    n_ = jnp.tanh(gi[:, 2 * d:] + r * gh[:, 2 * d:])
    new_mem = (1.0 - z) * n_ + z * mem
    upd = jnp.where(cnt > 0, new_mem, mem)
    upd_ref[...] = upd
    x_cat = jnp.concatenate([x_ref[...], upd], axis=1)
    q_ref[...] = jnp.dot(x_cat, wq[...], preferred_element_type=f32)
    k = jnp.dot(x_cat, wk[...], preferred_element_type=f32)
    v = jnp.dot(x_cat, wv[...], preferred_element_type=f32)
    kv_ref[...] = jnp.concatenate([k, v], axis=1)
    skip_ref[...] = jnp.dot(x_cat, wsk[...], preferred_element_type=f32)


def _tcb_edge_body(rel_ref, attr_ref, wt_ref, bt_ref, wet_ref, wea_ref, e_ref):
    cosv = jnp.cos(rel_ref[...] * wt_ref[...] + bt_ref[...])
    e_ref[...] = (jnp.dot(cosv, wet_ref[...], preferred_element_type=jnp.float32)
                  + jnp.dot(attr_ref[...], wea_ref[...],
                            preferred_element_type=jnp.float32))


def _tcd_body(a0_ref, a1_ref, skip_ref, wlin_ref, blin_ref, out_ref):
    acc = a0_ref[...] + a1_ref[...]
    denom = jnp.maximum(acc[:, 128:129], 1e-16)
    out = acc[:, :128] / denom + skip_ref[...]
    out_ref[...] = jnp.dot(out, wlin_ref[...],
                           preferred_element_type=jnp.float32) + blin_ref[...]


def kernel(event_type_ids, event_src_ids, event_dst_ids, event_embeddings,
           event_timestamps, x, edge_index, edge_attr, edge_last_update,
           batch, memory, emb_table, Wt, bt, W_ih, b_ih, W_hh, b_hh,
           Wq, Wk, Wv, We, Wskip, Wlin, blin):
    f32 = jnp.float32
    srcs = edge_index[0]
    dsts = edge_index[1]
    embt16 = jnp.concatenate([emb_table, jnp.ones((7, 1), f32),
                              jnp.zeros((7, 7), f32)], axis=1).reshape(112)
    W1 = W_ih[8:72]
    W2 = W_ih[72:136]
    W3 = W_ih[136:168]
    W4 = W_ih[168:296]
    W6 = jnp.concatenate([W_ih[0:8], jnp.zeros((8, 192), f32)], axis=0)
    WeT = We[:32]
    WeA = We[32:]

    time_embs = pl.pallas_call(
        _tc0_body,
        out_shape=jax.ShapeDtypeStruct((B, 32), f32),
    )(event_timestamps.reshape(B, 1), Wt, bt.reshape(1, 32))

    a1, a2 = _sc_a(event_src_ids, event_dst_ids, memory)
    (bb,) = _sc_b(event_src_ids, event_dst_ids, event_embeddings)
    c1, c2, rel_t = _sc_c(event_src_ids, event_dst_ids, event_type_ids,
                          time_embs, batch, event_timestamps, srcs,
                          edge_last_update, embt16)

    nb = 1000
    grid_n = N // nb
    row = lambda i: (i, 0)
    full = lambda i: (0, 0)
    upd, q, kv, skip = pl.pallas_call(
        _tcb_node_body,
        grid=(grid_n,),
        in_specs=[
            pl.BlockSpec((nb, 64), row), pl.BlockSpec((nb, 64), row),
            pl.BlockSpec((nb, 64), row), pl.BlockSpec((nb, 64), row),
            pl.BlockSpec((nb, 32), row), pl.BlockSpec((nb, 32), row),
            pl.BlockSpec((nb, 128), row), pl.BlockSpec((nb, 128), row),
            pl.BlockSpec((nb, 16), row), pl.BlockSpec((nb, 16), row),
            pl.BlockSpec((nb, 64), row), pl.BlockSpec((nb, 128), row),
            pl.BlockSpec((64, 192), full), pl.BlockSpec((64, 192), full),
            pl.BlockSpec((32, 192), full), pl.BlockSpec((128, 192), full),
            pl.BlockSpec((16, 192), full),
            pl.BlockSpec((1, 192), full), pl.BlockSpec((64, 192), full),
            pl.BlockSpec((1, 192), full),
            pl.BlockSpec((192, 128), full), pl.BlockSpec((192, 128), full),
            pl.BlockSpec((192, 128), full), pl.BlockSpec((192, 128), full),
        ],
        out_specs=[
            pl.BlockSpec((nb, 64), row), pl.BlockSpec((nb, 128), row),
            pl.BlockSpec((nb, 256), row), pl.BlockSpec((nb, 128), row),
        ],
        out_shape=[
            jax.ShapeDtypeStruct((N, 64), f32),
            jax.ShapeDtypeStruct((N, 128), f32),
            jax.ShapeDtypeStruct((N, 256), f32),
            jax.ShapeDtypeStruct((N, 128), f32),
        ],
    )(a1[0, :N], a1[1, :N], a2[0, :N], a2[1, :N], c1[0, :N], c1[1, :N],
      bb[0, :N], bb[1, :N], c2[0, :N], c2[1, :N], memory, x,
      W1, W2, W3, W4, W6, b_ih.reshape(1, 192), W_hh, b_hh.reshape(1, 192),
      Wq, Wk, Wv, Wskip)

    ebk = 4000
    e_tab = pl.pallas_call(
        _tcb_edge_body,
        grid=(E // ebk,),
        in_specs=[
            pl.BlockSpec((ebk, 1), row), pl.BlockSpec((ebk, 128), row),
            pl.BlockSpec((1, 32), full), pl.BlockSpec((1, 32), full),
            pl.BlockSpec((32, 128), full), pl.BlockSpec((128, 128), full),
        ],
        out_specs=pl.BlockSpec((ebk, 128), row),
        out_shape=jax.ShapeDtypeStruct((E, 128), f32),
    )(rel_t.reshape(E, 1), edge_attr, Wt, bt.reshape(1, 32), WeT, WeA)

    (acc_parts,) = _sc_d(srcs, dsts.reshape(E // 32, 32), kv, q, e_tab)

    node_embeddings = pl.pallas_call(
        _tcd_body,
        grid=(grid_n,),
        in_specs=[
            pl.BlockSpec((nb, 144), row), pl.BlockSpec((nb, 144), row),
            pl.BlockSpec((nb, 128), row),
            pl.BlockSpec((128, 128), full), pl.BlockSpec((1, 128), full),
        ],
        out_specs=pl.BlockSpec((nb, 128), row),
        out_shape=jax.ShapeDtypeStruct((N, 128), f32),
    )(acc_parts[0, :N], acc_parts[1, :N], skip, Wlin, blin.reshape(1, 128))

    return node_embeddings, upd


# speedup vs baseline: 5.8752x; 1.0689x over previous
import functools

import jax
import jax.numpy as jnp
from jax import lax
from jax.experimental import pallas as pl
from jax.experimental.pallas import tpu as pltpu
from jax.experimental.pallas import tpu_sc as plsc

N = 10000
NP = 10112
E = 320000
B = 4096
NC = 2
NS = 16
NW = NC * NS

EV_CHUNK = 128
E_PER_TILE = E // NW
E_CHUNK = 32
E_TAIL = E_PER_TILE % E_CHUNK
R_CHUNK = 2000
SLABS = (128, 128, 128, 128, 120)

_mesh = plsc.VectorSubcoreMesh(core_axis_name="c", subcore_axis_name="s")
_cp = pltpu.CompilerParams(needs_layout_passes=False, use_tc_tiling_on_sc=False)


def _zero2d(ref, rows, cols):
    def body(i, _):
        for c in range(cols // 16):
            ref[i, pl.ds(c * 16, 16)] = jnp.zeros((16,), jnp.float32)
        return 0
    lax.fori_loop(0, rows, body, 0)


def _fill_table(zbuf, table, sid):
    for j, zr in enumerate(SLABS):
        pltpu.sync_copy(zbuf.at[pl.ds(0, zr)],
                        table.at[pl.ds(sid * 632 + j * 128, zr)])


def _dump_table(table, out, cid, sid):
    for j, zr in enumerate(SLABS):
        r0 = sid * 632 + j * 128
        pltpu.sync_copy(table.at[pl.ds(r0, zr)], out.at[cid, pl.ds(r0, zr)])


@functools.partial(
    pl.kernel, mesh=_mesh, compiler_params=_cp,
    out_type=[
        jax.ShapeDtypeStruct((NC, NP, 64), jnp.float32),
        jax.ShapeDtypeStruct((NC, NP, 64), jnp.float32),
    ],
    scratch_types=[
        pltpu.VMEM_SHARED((NP, 64), jnp.float32),
        pltpu.VMEM_SHARED((NP, 64), jnp.float32),
        pltpu.VMEM((EV_CHUNK, 64), jnp.float32),
        pltpu.VMEM((EV_CHUNK, 64), jnp.float32),
        pltpu.VMEM((EV_CHUNK,), jnp.int32),
        pltpu.VMEM((EV_CHUNK,), jnp.int32),
        pltpu.SemaphoreType.DMA,
        pltpu.SemaphoreType.DMA,
    ],
)
def _sc_a(src_ids, dst_ids, mem_hbm, o1, o2,
          s1, s2, smem_v, dmem_v, idxs_v, idxd_v, sem1, sem2):
    cid = lax.axis_index("c")
    sid = lax.axis_index("s")
    _zero2d(smem_v, EV_CHUNK, 64)
    _fill_table(smem_v, s1, sid)
    _fill_table(smem_v, s2, sid)
    plsc.subcore_barrier()

    ev0 = cid * (B // NC) + sid * EV_CHUNK
    pltpu.sync_copy(src_ids.at[pl.ds(ev0, EV_CHUNK)], idxs_v)
    pltpu.sync_copy(dst_ids.at[pl.ds(ev0, EV_CHUNK)], idxd_v)
    cp1 = pltpu.async_copy(mem_hbm.at[idxs_v], smem_v, sem1)
    cp2 = pltpu.async_copy(mem_hbm.at[idxd_v], dmem_v, sem2)
    cp1.wait()
    cp2.wait()
    pltpu.sync_copy(smem_v, s1.at[idxs_v], add=True)
    pltpu.sync_copy(smem_v, s1.at[idxd_v], add=True)
    pltpu.sync_copy(dmem_v, s2.at[idxs_v], add=True)
    pltpu.sync_copy(dmem_v, s2.at[idxd_v], add=True)

    plsc.subcore_barrier()
    _dump_table(s1, o1, cid, sid)
    _dump_table(s2, o2, cid, sid)


@functools.partial(
    pl.kernel, mesh=_mesh, compiler_params=_cp,
    out_type=[jax.ShapeDtypeStruct((NC, NP, 128), jnp.float32)],
    scratch_types=[
        pltpu.VMEM_SHARED((NP, 128), jnp.float32),
        pltpu.VMEM((EV_CHUNK, 128), jnp.float32),
        pltpu.VMEM((EV_CHUNK,), jnp.int32),
        pltpu.VMEM((EV_CHUNK,), jnp.int32),
    ],
)
def _sc_b(src_ids, dst_ids, eemb_hbm, o1, s1, ebuf_v, idxs_v, idxd_v):
    cid = lax.axis_index("c")
    sid = lax.axis_index("s")
    _zero2d(ebuf_v, EV_CHUNK, 128)
    _fill_table(ebuf_v, s1, sid)
    plsc.subcore_barrier()

    ev0 = cid * (B // NC) + sid * EV_CHUNK
    pltpu.sync_copy(src_ids.at[pl.ds(ev0, EV_CHUNK)], idxs_v)
    pltpu.sync_copy(dst_ids.at[pl.ds(ev0, EV_CHUNK)], idxd_v)
    pltpu.sync_copy(eemb_hbm.at[pl.ds(ev0, EV_CHUNK)], ebuf_v)
    pltpu.sync_copy(ebuf_v, s1.at[idxs_v], add=True)
    pltpu.sync_copy(ebuf_v, s1.at[idxd_v], add=True)

    plsc.subcore_barrier()
    _dump_table(s1, o1, cid, sid)


@functools.partial(
    pl.kernel, mesh=_mesh, compiler_params=_cp,
    out_type=[
        jax.ShapeDtypeStruct((NC, NP, 32), jnp.float32),
        jax.ShapeDtypeStruct((NC, NP, 16), jnp.float32),
        jax.ShapeDtypeStruct((E,), jnp.float32),
    ],
    scratch_types=[
        pltpu.VMEM_SHARED((NP, 32), jnp.float32),
        pltpu.VMEM_SHARED((NP, 16), jnp.float32),
        pltpu.VMEM((EV_CHUNK, 32), jnp.float32),
        pltpu.VMEM((EV_CHUNK, 16), jnp.float32),
        pltpu.VMEM((EV_CHUNK,), jnp.int32),
        pltpu.VMEM((EV_CHUNK,), jnp.int32),
        pltpu.VMEM((EV_CHUNK,), jnp.int32),
        pltpu.VMEM((112,), jnp.float32),
        pltpu.VMEM((B,), jnp.float32),
        pltpu.VMEM((R_CHUNK,), jnp.int32),
        pltpu.VMEM((N,), jnp.float32),
        pltpu.VMEM((R_CHUNK,), jnp.int32),
        pltpu.VMEM((R_CHUNK,), jnp.float32),
        pltpu.VMEM((R_CHUNK,), jnp.float32),
    ],
)
def _sc_c(src_ids, dst_ids, type_ids, time_embs, batch_hbm, ts_hbm,
          esrc_hbm, elu_hbm, embt_hbm,
          o1, o2, relt_out,
          s1, s2, tbuf_v, tybuf_v, idxs_v, idxd_v, tids_v, embt_v,
          tsv, bch_v, nts_v, sbuf_v, elub_v, rbuf_v):
    cid = lax.axis_index("c")
    sid = lax.axis_index("s")
    wid = sid * NC + cid
    _zero2d(tbuf_v, EV_CHUNK, 32)
    _fill_table(tbuf_v, s1, sid)
    _zero2d(tybuf_v, EV_CHUNK, 16)
    _fill_table(tybuf_v, s2, sid)
    plsc.subcore_barrier()

    pltpu.sync_copy(embt_hbm, embt_v)
    ev0 = cid * (B // NC) + sid * EV_CHUNK
    pltpu.sync_copy(src_ids.at[pl.ds(ev0, EV_CHUNK)], idxs_v)
    pltpu.sync_copy(dst_ids.at[pl.ds(ev0, EV_CHUNK)], idxd_v)
    pltpu.sync_copy(time_embs.at[pl.ds(ev0, EV_CHUNK)], tbuf_v)
    pltpu.sync_copy(type_ids.at[pl.ds(ev0, EV_CHUNK)], tids_v)
    lanes = lax.iota(jnp.int32, 16)

    def tyb(g, _):
        tv = tids_v[pl.ds(g * 16, 16)]
        for j in range(16):
            ti = tv[j]
            vals = plsc.load_gather(embt_v, [ti * 16 + lanes])
            tybuf_v[g * 16 + j, pl.ds(0, 16)] = vals
        return 0
    lax.fori_loop(0, EV_CHUNK // 16, tyb, 0)

    pltpu.sync_copy(tbuf_v, s1.at[idxs_v], add=True)
    pltpu.sync_copy(tbuf_v, s1.at[idxd_v], add=True)
    pltpu.sync_copy(tybuf_v, s2.at[idxs_v], add=True)
    pltpu.sync_copy(tybuf_v, s2.at[idxd_v], add=True)

    pltpu.sync_copy(ts_hbm, tsv)
    for m in range(N // R_CHUNK):
        pltpu.sync_copy(batch_hbm.at[pl.ds(m * R_CHUNK, R_CHUNK)], bch_v)

        def nbody(j, _):
            idx = bch_v[pl.ds(j * 16, 16)]
            nts_v[pl.ds(m * R_CHUNK + j * 16, 16)] = plsc.load_gather(tsv, [idx])
            return 0
        lax.fori_loop(0, R_CHUNK // 16, nbody, 0)

    for t in range(E_PER_TILE // R_CHUNK):
        eb = wid * E_PER_TILE + t * R_CHUNK
        pltpu.sync_copy(esrc_hbm.at[pl.ds(eb, R_CHUNK)], sbuf_v)
        pltpu.sync_copy(elu_hbm.at[pl.ds(eb, R_CHUNK)], elub_v)

        def rbody(j, _):
            ii = sbuf_v[pl.ds(j * 16, 16)]
            tse = plsc.load_gather(nts_v, [ii])
            rbuf_v[pl.ds(j * 16, 16)] = tse - elub_v[pl.ds(j * 16, 16)]
            return 0
        lax.fori_loop(0, R_CHUNK // 16, rbody, 0)
        pltpu.sync_copy(rbuf_v, relt_out.at[pl.ds(eb, R_CHUNK)])

    plsc.subcore_barrier()
    _dump_table(s1, o1, cid, sid)
    _dump_table(s2, o2, cid, sid)


E_BLK = 26
N_BLK = 12


@functools.partial(
    pl.kernel, mesh=_mesh, compiler_params=_cp,
    out_type=[jax.ShapeDtypeStruct((NC, NP, 144), jnp.float32)],
    scratch_types=[
        pltpu.VMEM_SHARED((NP, 144), jnp.float32),
        pltpu.VMEM((E_BLK * E_CHUNK,), jnp.int32),
        pltpu.VMEM((E_BLK, E_CHUNK), jnp.int32),
        pltpu.VMEM((2, E_CHUNK, 256), jnp.float32),
        pltpu.VMEM((2, E_CHUNK, 128), jnp.float32),
        pltpu.VMEM((2, E_CHUNK, 128), jnp.float32),
        pltpu.VMEM((E_CHUNK, 144), jnp.float32),
        pltpu.VMEM((E_CHUNK,), jnp.int32),
        pltpu.VMEM((1, E_CHUNK), jnp.int32),
        pltpu.SemaphoreType.DMA,
        pltpu.SemaphoreType.DMA,
    ],
)
def _sc_d(esrc_hbm, edst2_hbm, kv_hbm, q_hbm, etab_hbm,
          acc_out,
          acc, sblk_v, dblk_v, kvb_v, qb_v, eb_v, rowsb_v,
          tidxs_v, tdx_v, sem0, sem1):
    cid = lax.axis_index("c")
    sid = lax.axis_index("s")
    wid = sid * NC + cid
    sems = (sem0, sem1)

    _zero2d(rowsb_v, E_CHUNK, 144)
    nz = 632 // E_CHUNK
    for j in range(nz):
        pltpu.sync_copy(rowsb_v, acc.at[pl.ds(sid * 632 + j * E_CHUNK, E_CHUNK)])
    if 632 % E_CHUNK:
        pltpu.sync_copy(rowsb_v.at[pl.ds(0, 632 % E_CHUNK)],
                        acc.at[pl.ds(sid * 632 + nz * E_CHUNK, 632 % E_CHUNK)])
    plsc.subcore_barrier()

    inv_sqrt = jnp.float32(128.0 ** -0.5)
    lanes = lax.iota(jnp.int32, 16)
    umask = jnp.where(lanes == 0, jnp.float32(1.0), jnp.float32(0.0))

    def start(slot, brow, k):
        pltpu.async_copy(kv_hbm.at[sblk_v.at[pl.ds(k * E_CHUNK, E_CHUNK)]],
                         kvb_v.at[slot], sems[slot])
        pltpu.async_copy(q_hbm.at[dblk_v.at[k]], qb_v.at[slot], sems[slot])
        pltpu.async_copy(etab_hbm.at[pl.ds((brow + k) * E_CHUNK, E_CHUNK)],
                         eb_v.at[slot], sems[slot])

    def drain(slot, brow, k):
        pltpu.make_async_copy(kv_hbm.at[sblk_v.at[pl.ds(k * E_CHUNK, E_CHUNK)]],
                              kvb_v.at[slot], sems[slot]).wait()
        pltpu.make_async_copy(q_hbm.at[dblk_v.at[k]], qb_v.at[slot], sems[slot]).wait()
        pltpu.make_async_copy(etab_hbm.at[pl.ds((brow + k) * E_CHUNK, E_CHUNK)],
                              eb_v.at[slot], sems[slot]).wait()

    def compute(slot, idx_ref):
        def edge(i, _):
            d16 = jnp.zeros((16,), jnp.float32)
            for c in range(8):
                qv = qb_v[slot, i, pl.ds(c * 16, 16)]
                kvv = kvb_v[slot, i, pl.ds(c * 16, 16)]
                ev = eb_v[slot, i, pl.ds(c * 16, 16)]
                d16 = d16 + qv * (kvv + ev)
            s = jnp.sum(d16) * inv_sqrt
            svec = jnp.exp(jnp.full((16,), s, jnp.float32))
            for c in range(8):
                vv = kvb_v[slot, i, pl.ds(128 + c * 16, 16)]
                ev = eb_v[slot, i, pl.ds(c * 16, 16)]
                rowsb_v[i, pl.ds(c * 16, 16)] = svec * (vv + ev)
            rowsb_v[i, pl.ds(128, 16)] = svec * umask
            return 0
        lax.fori_loop(0, E_CHUNK, edge, 0)
        pltpu.sync_copy(rowsb_v, acc.at[idx_ref], add=True)

    def block(blk, _):
        brow = (wid * N_BLK + blk) * E_BLK
        pltpu.sync_copy(esrc_hbm.at[pl.ds(brow * E_CHUNK, E_BLK * E_CHUNK)], sblk_v)
        pltpu.sync_copy(edst2_hbm.at[pl.ds(brow, E_BLK)], dblk_v)
        start(0, brow, 0)

        def pair(j, _):
            start(1, brow, 2 * j + 1)
            drain(0, brow, 2 * j)
            compute(0, dblk_v.at[2 * j])

            @pl.when(2 * j + 2 < E_BLK)
            def _():
                start(0, brow, 2 * j + 2)
            drain(1, brow, 2 * j + 1)
            compute(1, dblk_v.at[2 * j + 1])
            return 0
        lax.fori_loop(0, E_BLK // 2, pair, 0)
        return 0
    lax.fori_loop(0, N_BLK, block, 0)

    @pl.when(wid < 16)
    def _():
        trow = NW * N_BLK * E_BLK + wid
        pltpu.sync_copy(esrc_hbm.at[pl.ds(trow * E_CHUNK, E_CHUNK)], tidxs_v)
        pltpu.sync_copy(edst2_hbm.at[pl.ds(trow, 1)], tdx_v)
        tc1 = pltpu.async_copy(kv_hbm.at[tidxs_v], kvb_v.at[0], sem0)
        tc2 = pltpu.async_copy(q_hbm.at[tdx_v.at[0]], qb_v.at[0], sem0)
        tc3 = pltpu.async_copy(etab_hbm.at[pl.ds(trow * E_CHUNK, E_CHUNK)],
                               eb_v.at[0], sem0)
        tc1.wait()
        tc2.wait()
        tc3.wait()
        compute(0, tdx_v.at[0])

    plsc.subcore_barrier()
    _dump_table(acc, acc_out, cid, sid)


def _tc0_body(ts_ref, wt_ref, bt_ref, out_ref):
    out_ref[...] = jnp.cos(ts_ref[...] * wt_ref[...] + bt_ref[...])


def _tcb_node_body(ms0, ms1, md0, md1, mt0, mt1, me0, me1, mty0, mty1,
                   mem_ref, x_ref, w1, w2, w3, w4, w6, bih, whh, bhh,
                   wq, wk, wv, wsk,
                   upd_ref, q_ref, kv_ref, skip_ref):
    f32 = jnp.float32
    ms = ms0[...] + ms1[...]
    md = md0[...] + md1[...]
    mt = mt0[...] + mt1[...]
    me = me0[...] + me1[...]
    mty = mty0[...] + mty1[...]
    cnt = mty[:, 8:9]
    inv = 1.0 / jnp.maximum(cnt, 1.0)
    gi = (jnp.dot(ms * inv, w1[...], preferred_element_type=f32)
          + jnp.dot(md * inv, w2[...], preferred_element_type=f32)
          + jnp.dot(mt * inv, w3[...], preferred_element_type=f32)
          + jnp.dot(me * inv, w4[...], preferred_element_type=f32)
          + jnp.dot(mty * inv, w6[...], preferred_element_type=f32)
          + bih[...])
    mem = mem_ref[...]
    gh = jnp.dot(mem, whh[...], preferred_element_type=f32) + bhh[...]
    d = 64
    r = jax.nn.sigmoid(gi[:, :d] + gh[:, :d])
    z = jax.nn.sigmoid(gi[:, d:2 * d] + gh[:, d:2 * d])
    n_ = jnp.tanh(gi[:, 2 * d:] + r * gh[:, 2 * d:])
    new_mem = (1.0 - z) * n_ + z * mem
    upd = jnp.where(cnt > 0, new_mem, mem)
    upd_ref[...] = upd
    x_cat = jnp.concatenate([x_ref[...], upd], axis=1)
    q_ref[...] = jnp.dot(x_cat, wq[...], preferred_element_type=f32)
    k = jnp.dot(x_cat, wk[...], preferred_element_type=f32)
    v = jnp.dot(x_cat, wv[...], preferred_element_type=f32)
    kv_ref[...] = jnp.concatenate([k, v], axis=1)
    skip_ref[...] = jnp.dot(x_cat, wsk[...], preferred_element_type=f32)


def _tcb_edge_body(rel_ref, attr_ref, wt_ref, bt_ref, wet_ref, wea_ref, e_ref):
    cosv = jnp.cos(rel_ref[0, 0][:, None] * wt_ref[...] + bt_ref[...])
    e_ref[...] = (jnp.dot(cosv, wet_ref[...], preferred_element_type=jnp.float32)
                  + jnp.dot(attr_ref[...], wea_ref[...],
                            preferred_element_type=jnp.float32))


def _tcd_body(a0_ref, a1_ref, skip_ref, wlin_ref, blin_ref, out_ref):
    acc = a0_ref[...] + a1_ref[...]
    denom = jnp.maximum(acc[:, 128:129], 1e-16)
    out = acc[:, :128] / denom + skip_ref[...]
    out_ref[...] = jnp.dot(out, wlin_ref[...],
                           preferred_element_type=jnp.float32) + blin_ref[...]


def kernel(event_type_ids, event_src_ids, event_dst_ids, event_embeddings,
           event_timestamps, x, edge_index, edge_attr, edge_last_update,
           batch, memory, emb_table, Wt, bt, W_ih, b_ih, W_hh, b_hh,
           Wq, Wk, Wv, We, Wskip, Wlin, blin):
    f32 = jnp.float32
    srcs = edge_index[0]
    dsts = edge_index[1]
    embt16 = jnp.concatenate([emb_table, jnp.ones((7, 1), f32),
                              jnp.zeros((7, 7), f32)], axis=1).reshape(112)
    W1 = W_ih[8:72]
    W2 = W_ih[72:136]
    W3 = W_ih[136:168]
    W4 = W_ih[168:296]
    W6 = jnp.concatenate([W_ih[0:8], jnp.zeros((8, 192), f32)], axis=0)
    WeT = We[:32]
    WeA = We[32:]

    time_embs = pl.pallas_call(
        _tc0_body,
        out_shape=jax.ShapeDtypeStruct((B, 32), f32),
    )(event_timestamps.reshape(B, 1), Wt, bt.reshape(1, 32))

    a1, a2 = _sc_a(event_src_ids, event_dst_ids, memory)
    (bb,) = _sc_b(event_src_ids, event_dst_ids, event_embeddings)
    c1, c2, rel_t = _sc_c(event_src_ids, event_dst_ids, event_type_ids,
                          time_embs, batch, event_timestamps, srcs,
                          edge_last_update, embt16)

    nb = 1000
    grid_n = N // nb
    row = lambda i: (i, 0)
    full = lambda i: (0, 0)
    upd, q, kv, skip = pl.pallas_call(
        _tcb_node_body,
        grid=(grid_n,),
        in_specs=[
            pl.BlockSpec((nb, 64), row), pl.BlockSpec((nb, 64), row),
            pl.BlockSpec((nb, 64), row), pl.BlockSpec((nb, 64), row),
            pl.BlockSpec((nb, 32), row), pl.BlockSpec((nb, 32), row),
            pl.BlockSpec((nb, 128), row), pl.BlockSpec((nb, 128), row),
            pl.BlockSpec((nb, 16), row), pl.BlockSpec((nb, 16), row),
            pl.BlockSpec((nb, 64), row), pl.BlockSpec((nb, 128), row),
            pl.BlockSpec((64, 192), full), pl.BlockSpec((64, 192), full),
            pl.BlockSpec((32, 192), full), pl.BlockSpec((128, 192), full),
            pl.BlockSpec((16, 192), full),
            pl.BlockSpec((1, 192), full), pl.BlockSpec((64, 192), full),
            pl.BlockSpec((1, 192), full),
            pl.BlockSpec((192, 128), full), pl.BlockSpec((192, 128), full),
            pl.BlockSpec((192, 128), full), pl.BlockSpec((192, 128), full),
        ],
        out_specs=[
            pl.BlockSpec((nb, 64), row), pl.BlockSpec((nb, 128), row),
            pl.BlockSpec((nb, 256), row), pl.BlockSpec((nb, 128), row),
        ],
        out_shape=[
            jax.ShapeDtypeStruct((N, 64), f32),
            jax.ShapeDtypeStruct((N, 128), f32),
            jax.ShapeDtypeStruct((N, 256), f32),
            jax.ShapeDtypeStruct((N, 128), f32),
        ],
    )(a1[0, :N], a1[1, :N], a2[0, :N], a2[1, :N], c1[0, :N], c1[1, :N],
      bb[0, :N], bb[1, :N], c2[0, :N], c2[1, :N], memory, x,
      W1, W2, W3, W4, W6, b_ih.reshape(1, 192), W_hh, b_hh.reshape(1, 192),
      Wq, Wk, Wv, Wskip)

    ebk = 4000
    e_tab = pl.pallas_call(
        _tcb_edge_body,
        grid=(E // ebk,),
        in_specs=[
            pl.BlockSpec((1, 1, ebk), lambda i: (i, 0, 0)), pl.BlockSpec((ebk, 128), row),
            pl.BlockSpec((1, 32), full), pl.BlockSpec((1, 32), full),
            pl.BlockSpec((32, 128), full), pl.BlockSpec((128, 128), full),
        ],
        out_specs=pl.BlockSpec((ebk, 128), row),
        out_shape=jax.ShapeDtypeStruct((E, 128), f32),
    )(rel_t.reshape(E // ebk, 1, ebk), edge_attr, Wt, bt.reshape(1, 32), WeT, WeA)

    (acc_parts,) = _sc_d(srcs, dsts.reshape(E // 32, 32), kv, q, e_tab)

    node_embeddings = pl.pallas_call(
        _tcd_body,
        grid=(grid_n,),
        in_specs=[
            pl.BlockSpec((nb, 144), row), pl.BlockSpec((nb, 144), row),
            pl.BlockSpec((nb, 128), row),
            pl.BlockSpec((128, 128), full), pl.BlockSpec((1, 128), full),
        ],
        out_specs=pl.BlockSpec((nb, 128), row),
        out_shape=jax.ShapeDtypeStruct((N, 128), f32),
    )(acc_parts[0, :N], acc_parts[1, :N], skip, Wlin, blin.reshape(1, 128))

    return node_embeddings, upd


# speedup vs baseline: 7.3589x; 1.2525x over previous
import functools

import jax
import jax.numpy as jnp
from jax import lax
from jax.experimental import pallas as pl
from jax.experimental.pallas import tpu as pltpu
from jax.experimental.pallas import tpu_sc as plsc

N = 10000
NP = 10112
E = 320000
B = 4096
NC = 2
NS = 16
NW = NC * NS

EV_CHUNK = 128
E_PER_TILE = E // NW
E_CHUNK = 32
E_TAIL = E_PER_TILE % E_CHUNK
R_CHUNK = 2000
SLABS = (128, 128, 128, 128, 120)

_mesh = plsc.VectorSubcoreMesh(core_axis_name="c", subcore_axis_name="s")
_cp = pltpu.CompilerParams(needs_layout_passes=False, use_tc_tiling_on_sc=False)


def _zero2d(ref, rows, cols):
    def body(i, _):
        for c in range(cols // 16):
            ref[i, pl.ds(c * 16, 16)] = jnp.zeros((16,), jnp.float32)
        return 0
    lax.fori_loop(0, rows, body, 0)


def _fill_table(zbuf, table, sid):
    for j, zr in enumerate(SLABS):
        pltpu.sync_copy(zbuf.at[pl.ds(0, zr)],
                        table.at[pl.ds(sid * 632 + j * 128, zr)])


def _dump_table(table, out, cid, sid):
    for j, zr in enumerate(SLABS):
        r0 = sid * 632 + j * 128
        pltpu.sync_copy(table.at[pl.ds(r0, zr)], out.at[cid, pl.ds(r0, zr)])


@functools.partial(
    pl.kernel, mesh=_mesh, compiler_params=_cp,
    out_type=[
        jax.ShapeDtypeStruct((NC, NP, 64), jnp.float32),
        jax.ShapeDtypeStruct((NC, NP, 64), jnp.float32),
    ],
    scratch_types=[
        pltpu.VMEM_SHARED((NP, 64), jnp.float32),
        pltpu.VMEM_SHARED((NP, 64), jnp.float32),
        pltpu.VMEM((EV_CHUNK, 64), jnp.float32),
        pltpu.VMEM((EV_CHUNK, 64), jnp.float32),
        pltpu.VMEM((EV_CHUNK,), jnp.int32),
        pltpu.VMEM((EV_CHUNK,), jnp.int32),
        pltpu.SemaphoreType.DMA,
        pltpu.SemaphoreType.DMA,
    ],
)
def _sc_a(src_ids, dst_ids, mem_hbm, o1, o2,
          s1, s2, smem_v, dmem_v, idxs_v, idxd_v, sem1, sem2):
    cid = lax.axis_index("c")
    sid = lax.axis_index("s")
    _zero2d(smem_v, EV_CHUNK, 64)
    _fill_table(smem_v, s1, sid)
    _fill_table(smem_v, s2, sid)
    plsc.subcore_barrier()

    ev0 = cid * (B // NC) + sid * EV_CHUNK
    pltpu.sync_copy(src_ids.at[pl.ds(ev0, EV_CHUNK)], idxs_v)
    pltpu.sync_copy(dst_ids.at[pl.ds(ev0, EV_CHUNK)], idxd_v)
    cp1 = pltpu.async_copy(mem_hbm.at[idxs_v], smem_v, sem1)
    cp2 = pltpu.async_copy(mem_hbm.at[idxd_v], dmem_v, sem2)
    cp1.wait()
    cp2.wait()
    pltpu.sync_copy(smem_v, s1.at[idxs_v], add=True)
    pltpu.sync_copy(smem_v, s1.at[idxd_v], add=True)
    pltpu.sync_copy(dmem_v, s2.at[idxs_v], add=True)
    pltpu.sync_copy(dmem_v, s2.at[idxd_v], add=True)

    plsc.subcore_barrier()
    _dump_table(s1, o1, cid, sid)
    _dump_table(s2, o2, cid, sid)


@functools.partial(
    pl.kernel, mesh=_mesh, compiler_params=_cp,
    out_type=[jax.ShapeDtypeStruct((NC, NP, 128), jnp.float32)],
    scratch_types=[
        pltpu.VMEM_SHARED((NP, 128), jnp.float32),
        pltpu.VMEM((EV_CHUNK, 128), jnp.float32),
        pltpu.VMEM((EV_CHUNK,), jnp.int32),
        pltpu.VMEM((EV_CHUNK,), jnp.int32),
    ],
)
def _sc_b(src_ids, dst_ids, eemb_hbm, o1, s1, ebuf_v, idxs_v, idxd_v):
    cid = lax.axis_index("c")
    sid = lax.axis_index("s")
    _zero2d(ebuf_v, EV_CHUNK, 128)
    _fill_table(ebuf_v, s1, sid)
    plsc.subcore_barrier()

    ev0 = cid * (B // NC) + sid * EV_CHUNK
    pltpu.sync_copy(src_ids.at[pl.ds(ev0, EV_CHUNK)], idxs_v)
    pltpu.sync_copy(dst_ids.at[pl.ds(ev0, EV_CHUNK)], idxd_v)
    pltpu.sync_copy(eemb_hbm.at[pl.ds(ev0, EV_CHUNK)], ebuf_v)
    pltpu.sync_copy(ebuf_v, s1.at[idxs_v], add=True)
    pltpu.sync_copy(ebuf_v, s1.at[idxd_v], add=True)

    plsc.subcore_barrier()
    _dump_table(s1, o1, cid, sid)


@functools.partial(
    pl.kernel, mesh=_mesh, compiler_params=_cp,
    out_type=[
        jax.ShapeDtypeStruct((NC, NP, 32), jnp.float32),
        jax.ShapeDtypeStruct((NC, NP, 16), jnp.float32),
        jax.ShapeDtypeStruct((E,), jnp.float32),
    ],
    scratch_types=[
        pltpu.VMEM_SHARED((NP, 32), jnp.float32),
        pltpu.VMEM_SHARED((NP, 16), jnp.float32),
        pltpu.VMEM((EV_CHUNK, 32), jnp.float32),
        pltpu.VMEM((EV_CHUNK, 16), jnp.float32),
        pltpu.VMEM((EV_CHUNK,), jnp.int32),
        pltpu.VMEM((EV_CHUNK,), jnp.int32),
        pltpu.VMEM((EV_CHUNK,), jnp.int32),
        pltpu.VMEM((112,), jnp.float32),
        pltpu.VMEM((B,), jnp.float32),
        pltpu.VMEM((R_CHUNK,), jnp.int32),
        pltpu.VMEM((N,), jnp.float32),
        pltpu.VMEM((R_CHUNK,), jnp.int32),
        pltpu.VMEM((R_CHUNK,), jnp.float32),
        pltpu.VMEM((R_CHUNK,), jnp.float32),
    ],
)
def _sc_c(src_ids, dst_ids, type_ids, time_embs, batch_hbm, ts_hbm,
          esrc_hbm, elu_hbm, embt_hbm,
          o1, o2, relt_out,
          s1, s2, tbuf_v, tybuf_v, idxs_v, idxd_v, tids_v, embt_v,
          tsv, bch_v, nts_v, sbuf_v, elub_v, rbuf_v):
    cid = lax.axis_index("c")
    sid = lax.axis_index("s")
    wid = sid * NC + cid
    _zero2d(tbuf_v, EV_CHUNK, 32)
    _fill_table(tbuf_v, s1, sid)
    _zero2d(tybuf_v, EV_CHUNK, 16)
    _fill_table(tybuf_v, s2, sid)
    plsc.subcore_barrier()

    pltpu.sync_copy(embt_hbm, embt_v)
    ev0 = cid * (B // NC) + sid * EV_CHUNK
    pltpu.sync_copy(src_ids.at[pl.ds(ev0, EV_CHUNK)], idxs_v)
    pltpu.sync_copy(dst_ids.at[pl.ds(ev0, EV_CHUNK)], idxd_v)
    pltpu.sync_copy(time_embs.at[pl.ds(ev0, EV_CHUNK)], tbuf_v)
    pltpu.sync_copy(type_ids.at[pl.ds(ev0, EV_CHUNK)], tids_v)
    lanes = lax.iota(jnp.int32, 16)

    def tyb(g, _):
        tv = tids_v[pl.ds(g * 16, 16)]
        for j in range(16):
            ti = tv[j]
            vals = plsc.load_gather(embt_v, [ti * 16 + lanes])
            tybuf_v[g * 16 + j, pl.ds(0, 16)] = vals
        return 0
    lax.fori_loop(0, EV_CHUNK // 16, tyb, 0)

    pltpu.sync_copy(tbuf_v, s1.at[idxs_v], add=True)
    pltpu.sync_copy(tbuf_v, s1.at[idxd_v], add=True)
    pltpu.sync_copy(tybuf_v, s2.at[idxs_v], add=True)
    pltpu.sync_copy(tybuf_v, s2.at[idxd_v], add=True)

    pltpu.sync_copy(ts_hbm, tsv)
    for m in range(N // R_CHUNK):
        pltpu.sync_copy(batch_hbm.at[pl.ds(m * R_CHUNK, R_CHUNK)], bch_v)

        def nbody(j, _):
            idx = bch_v[pl.ds(j * 16, 16)]
            nts_v[pl.ds(m * R_CHUNK + j * 16, 16)] = plsc.load_gather(tsv, [idx])
            return 0
        lax.fori_loop(0, R_CHUNK // 16, nbody, 0)

    for t in range(E_PER_TILE // R_CHUNK):
        eb = wid * E_PER_TILE + t * R_CHUNK
        pltpu.sync_copy(esrc_hbm.at[pl.ds(eb, R_CHUNK)], sbuf_v)
        pltpu.sync_copy(elu_hbm.at[pl.ds(eb, R_CHUNK)], elub_v)

        def rbody(j, _):
            ii = sbuf_v[pl.ds(j * 16, 16)]
            tse = plsc.load_gather(nts_v, [ii])
            rbuf_v[pl.ds(j * 16, 16)] = tse - elub_v[pl.ds(j * 16, 16)]
            return 0
        lax.fori_loop(0, R_CHUNK // 16, rbody, 0)
        pltpu.sync_copy(rbuf_v, relt_out.at[pl.ds(eb, R_CHUNK)])

    plsc.subcore_barrier()
    _dump_table(s1, o1, cid, sid)
    _dump_table(s2, o2, cid, sid)


E_BLK = 26
N_BLK = 12


@functools.partial(
    pl.kernel, mesh=_mesh, compiler_params=_cp,
    out_type=[jax.ShapeDtypeStruct((NC, NP, 144), jnp.float32)],
    scratch_types=[
        pltpu.VMEM_SHARED((NP, 144), jnp.float32),
        pltpu.VMEM((E_BLK * E_CHUNK,), jnp.int32),
        pltpu.VMEM((E_BLK, E_CHUNK), jnp.int32),
        pltpu.VMEM((2, E_CHUNK, 256), jnp.float32),
        pltpu.VMEM((2, E_CHUNK, 128), jnp.float32),
        pltpu.VMEM((2, E_CHUNK, 128), jnp.float32),
        pltpu.VMEM((E_CHUNK, 144), jnp.float32),
        pltpu.VMEM((E_CHUNK,), jnp.int32),
        pltpu.VMEM((1, E_CHUNK), jnp.int32),
        pltpu.SemaphoreType.DMA,
        pltpu.SemaphoreType.DMA,
    ],
)
def _sc_d(esrc_hbm, edst2_hbm, kv_hbm, q_hbm, etab_hbm,
          acc_out,
          acc, sblk_v, dblk_v, kvb_v, qb_v, eb_v, rowsb_v,
          tidxs_v, tdx_v, sem0, sem1):
    cid = lax.axis_index("c")
    sid = lax.axis_index("s")
    wid = sid * NC + cid
    sems = (sem0, sem1)

    _zero2d(rowsb_v, E_CHUNK, 144)
    nz = 632 // E_CHUNK
    for j in range(nz):
        pltpu.sync_copy(rowsb_v, acc.at[pl.ds(sid * 632 + j * E_CHUNK, E_CHUNK)])
    if 632 % E_CHUNK:
        pltpu.sync_copy(rowsb_v.at[pl.ds(0, 632 % E_CHUNK)],
                        acc.at[pl.ds(sid * 632 + nz * E_CHUNK, 632 % E_CHUNK)])
    plsc.subcore_barrier()

    inv_sqrt = jnp.float32(128.0 ** -0.5)
    lanes = lax.iota(jnp.int32, 16)
    umask = jnp.where(lanes == 0, jnp.float32(1.0), jnp.float32(0.0))

    def start(slot, brow, k):
        pltpu.async_copy(kv_hbm.at[sblk_v.at[pl.ds(k * E_CHUNK, E_CHUNK)]],
                         kvb_v.at[slot], sems[slot])
        pltpu.async_copy(q_hbm.at[dblk_v.at[k]], qb_v.at[slot], sems[slot])
        pltpu.async_copy(etab_hbm.at[pl.ds((brow + k) * E_CHUNK, E_CHUNK)],
                         eb_v.at[slot], sems[slot])

    def drain(slot, brow, k):
        pltpu.make_async_copy(kv_hbm.at[sblk_v.at[pl.ds(k * E_CHUNK, E_CHUNK)]],
                              kvb_v.at[slot], sems[slot]).wait()
        pltpu.make_async_copy(q_hbm.at[dblk_v.at[k]], qb_v.at[slot], sems[slot]).wait()
        pltpu.make_async_copy(etab_hbm.at[pl.ds((brow + k) * E_CHUNK, E_CHUNK)],
                              eb_v.at[slot], sems[slot]).wait()

    def compute(slot, idx_ref):
        def edge(i, _):
            d16 = jnp.zeros((16,), jnp.float32)
            for c in range(8):
                qv = qb_v[slot, i, pl.ds(c * 16, 16)]
                kvv = kvb_v[slot, i, pl.ds(c * 16, 16)]
                ev = eb_v[slot, i, pl.ds(c * 16, 16)]
                d16 = d16 + qv * (kvv + ev)
            s = jnp.sum(d16) * inv_sqrt
            svec = jnp.exp(jnp.full((16,), s, jnp.float32))
            for c in range(8):
                vv = kvb_v[slot, i, pl.ds(128 + c * 16, 16)]
                ev = eb_v[slot, i, pl.ds(c * 16, 16)]
                rowsb_v[i, pl.ds(c * 16, 16)] = svec * (vv + ev)
            rowsb_v[i, pl.ds(128, 16)] = svec * umask
            return 0
        lax.fori_loop(0, E_CHUNK, edge, 0)
        pltpu.sync_copy(rowsb_v, acc.at[idx_ref], add=True)

    def block(blk, _):
        brow = (wid * N_BLK + blk) * E_BLK
        pltpu.sync_copy(esrc_hbm.at[pl.ds(brow * E_CHUNK, E_BLK * E_CHUNK)], sblk_v)
        pltpu.sync_copy(edst2_hbm.at[pl.ds(brow, E_BLK)], dblk_v)
        start(0, brow, 0)

        def pair(j, _):
            start(1, brow, 2 * j + 1)
            drain(0, brow, 2 * j)
            compute(0, dblk_v.at[2 * j])

            @pl.when(2 * j + 2 < E_BLK)
            def _():
                start(0, brow, 2 * j + 2)
            drain(1, brow, 2 * j + 1)
            compute(1, dblk_v.at[2 * j + 1])
            return 0
        lax.fori_loop(0, E_BLK // 2, pair, 0)
        return 0
    lax.fori_loop(0, N_BLK, block, 0)

    @pl.when(wid < 16)
    def _():
        trow = NW * N_BLK * E_BLK + wid
        pltpu.sync_copy(esrc_hbm.at[pl.ds(trow * E_CHUNK, E_CHUNK)], tidxs_v)
        pltpu.sync_copy(edst2_hbm.at[pl.ds(trow, 1)], tdx_v)
        tc1 = pltpu.async_copy(kv_hbm.at[tidxs_v], kvb_v.at[0], sem0)
        tc2 = pltpu.async_copy(q_hbm.at[tdx_v.at[0]], qb_v.at[0], sem0)
        tc3 = pltpu.async_copy(etab_hbm.at[pl.ds(trow * E_CHUNK, E_CHUNK)],
                               eb_v.at[0], sem0)
        tc1.wait()
        tc2.wait()
        tc3.wait()
        compute(0, tdx_v.at[0])

    plsc.subcore_barrier()
    _dump_table(acc, acc_out, cid, sid)


def _tc0_body(ts_ref, wt_ref, bt_ref, out_ref):
    out_ref[...] = jnp.cos(ts_ref[...] * wt_ref[...] + bt_ref[...])


def _tcb_node_body(ms0, ms1, md0, md1, mt0, mt1, me0, me1, mty0, mty1,
                   mem_ref, x_ref, w1, w2, w3, w4, w6, bih, whh, bhh,
                   wq, wk, wv, wsk,
                   upd_ref, q_ref, kv_ref, skip_ref):
    f32 = jnp.float32
    ms = ms0[...] + ms1[...]
    md = md0[...] + md1[...]
    mt = mt0[...] + mt1[...]
    me = me0[...] + me1[...]
    mty = mty0[...] + mty1[...]
    cnt = mty[:, 8:9]
    inv = 1.0 / jnp.maximum(cnt, 1.0)
    gi = (jnp.dot(ms * inv, w1[...], preferred_element_type=f32)
          + jnp.dot(md * inv, w2[...], preferred_element_type=f32)
          + jnp.dot(mt * inv, w3[...], preferred_element_type=f32)
          + jnp.dot(me * inv, w4[...], preferred_element_type=f32)
          + jnp.dot(mty * inv, w6[...], preferred_element_type=f32)
          + bih[...])
    mem = mem_ref[...]
    gh = jnp.dot(mem, whh[...], preferred_element_type=f32) + bhh[...]
    d = 64
    r = jax.nn.sigmoid(gi[:, :d] + gh[:, :d])
    z = jax.nn.sigmoid(gi[:, d:2 * d] + gh[:, d:2 * d])
    n_ = jnp.tanh(gi[:, 2 * d:] + r * gh[:, 2 * d:])
    new_mem = (1.0 - z) * n_ + z * mem
    upd = jnp.where(cnt > 0, new_mem, mem)
    upd_ref[...] = upd
    x_cat = jnp.concatenate([x_ref[...], upd], axis=1)
    q_ref[...] = jnp.dot(x_cat, wq[...], preferred_element_type=f32)
    k = jnp.dot(x_cat, wk[...], preferred_element_type=f32)
    v = jnp.dot(x_cat, wv[...], preferred_element_type=f32)
    kv_ref[...] = jnp.concatenate([k, v], axis=1)
    skip_ref[...] = jnp.dot(x_cat, wsk[...], preferred_element_type=f32)


def _tcb_edge_body(rel_ref, attr_ref, wtc_ref, btc_ref, wet_ref, wea_ref, e_ref):
    cosvT = jnp.cos(wtc_ref[...] * rel_ref[0, 0] + btc_ref[...])
    e_ref[...] = (
        jax.lax.dot_general(cosvT, wet_ref[...], (((0,), (0,)), ((), ())),
                            preferred_element_type=jnp.float32)
        + jnp.dot(attr_ref[...], wea_ref[...],
                  preferred_element_type=jnp.float32))


def _tcd_body(a0_ref, a1_ref, skip_ref, wlin_ref, blin_ref, out_ref):
    acc = a0_ref[...] + a1_ref[...]
    denom = jnp.maximum(acc[:, 128:129], 1e-16)
    out = acc[:, :128] / denom + skip_ref[...]
    out_ref[...] = jnp.dot(out, wlin_ref[...],
                           preferred_element_type=jnp.float32) + blin_ref[...]


def kernel(event_type_ids, event_src_ids, event_dst_ids, event_embeddings,
           event_timestamps, x, edge_index, edge_attr, edge_last_update,
           batch, memory, emb_table, Wt, bt, W_ih, b_ih, W_hh, b_hh,
           Wq, Wk, Wv, We, Wskip, Wlin, blin):
    f32 = jnp.float32
    srcs = edge_index[0]
    dsts = edge_index[1]
    embt16 = jnp.concatenate([emb_table, jnp.ones((7, 1), f32),
                              jnp.zeros((7, 7), f32)], axis=1).reshape(112)
    W1 = W_ih[8:72]
    W2 = W_ih[72:136]
    W3 = W_ih[136:168]
    W4 = W_ih[168:296]
    W6 = jnp.concatenate([W_ih[0:8], jnp.zeros((8, 192), f32)], axis=0)
    WeT = We[:32]
    WeA = We[32:]

    time_embs = pl.pallas_call(
        _tc0_body,
        out_shape=jax.ShapeDtypeStruct((B, 32), f32),
    )(event_timestamps.reshape(B, 1), Wt, bt.reshape(1, 32))

    a1, a2 = _sc_a(event_src_ids, event_dst_ids, memory)
    (bb,) = _sc_b(event_src_ids, event_dst_ids, event_embeddings)
    c1, c2, rel_t = _sc_c(event_src_ids, event_dst_ids, event_type_ids,
                          time_embs, batch, event_timestamps, srcs,
                          edge_last_update, embt16)

    nb = 1000
    grid_n = N // nb
    row = lambda i: (i, 0)
    full = lambda i: (0, 0)
    upd, q, kv, skip = pl.pallas_call(
        _tcb_node_body,
        grid=(grid_n,),
        in_specs=[
            pl.BlockSpec((nb, 64), row), pl.BlockSpec((nb, 64), row),
            pl.BlockSpec((nb, 64), row), pl.BlockSpec((nb, 64), row),
            pl.BlockSpec((nb, 32), row), pl.BlockSpec((nb, 32), row),
            pl.BlockSpec((nb, 128), row), pl.BlockSpec((nb, 128), row),
            pl.BlockSpec((nb, 16), row), pl.BlockSpec((nb, 16), row),
            pl.BlockSpec((nb, 64), row), pl.BlockSpec((nb, 128), row),
            pl.BlockSpec((64, 192), full), pl.BlockSpec((64, 192), full),
            pl.BlockSpec((32, 192), full), pl.BlockSpec((128, 192), full),
            pl.BlockSpec((16, 192), full),
            pl.BlockSpec((1, 192), full), pl.BlockSpec((64, 192), full),
            pl.BlockSpec((1, 192), full),
            pl.BlockSpec((192, 128), full), pl.BlockSpec((192, 128), full),
            pl.BlockSpec((192, 128), full), pl.BlockSpec((192, 128), full),
        ],
        out_specs=[
            pl.BlockSpec((nb, 64), row), pl.BlockSpec((nb, 128), row),
            pl.BlockSpec((nb, 256), row), pl.BlockSpec((nb, 128), row),
        ],
        out_shape=[
            jax.ShapeDtypeStruct((N, 64), f32),
            jax.ShapeDtypeStruct((N, 128), f32),
            jax.ShapeDtypeStruct((N, 256), f32),
            jax.ShapeDtypeStruct((N, 128), f32),
        ],
    )(a1[0, :N], a1[1, :N], a2[0, :N], a2[1, :N], c1[0, :N], c1[1, :N],
      bb[0, :N], bb[1, :N], c2[0, :N], c2[1, :N], memory, x,
      W1, W2, W3, W4, W6, b_ih.reshape(1, 192), W_hh, b_hh.reshape(1, 192),
      Wq, Wk, Wv, Wskip)

    ebk = 4000
    e_tab = pl.pallas_call(
        _tcb_edge_body,
        grid=(E // ebk,),
        in_specs=[
            pl.BlockSpec((1, 1, ebk), lambda i: (i, 0, 0)), pl.BlockSpec((ebk, 128), row),
            pl.BlockSpec((32, 1), full), pl.BlockSpec((32, 1), full),
            pl.BlockSpec((32, 128), full), pl.BlockSpec((128, 128), full),
        ],
        out_specs=pl.BlockSpec((ebk, 128), row),
        out_shape=jax.ShapeDtypeStruct((E, 128), f32),
    )(rel_t.reshape(E // ebk, 1, ebk), edge_attr, Wt.reshape(32, 1),
      bt.reshape(32, 1), WeT, WeA)

    (acc_parts,) = _sc_d(srcs, dsts.reshape(E // 32, 32), kv, q, e_tab)

    node_embeddings = pl.pallas_call(
        _tcd_body,
        grid=(grid_n,),
        in_specs=[
            pl.BlockSpec((nb, 144), row), pl.BlockSpec((nb, 144), row),
            pl.BlockSpec((nb, 128), row),
            pl.BlockSpec((128, 128), full), pl.BlockSpec((1, 128), full),
        ],
        out_specs=pl.BlockSpec((nb, 128), row),
        out_shape=jax.ShapeDtypeStruct((N, 128), f32),
    )(acc_parts[0, :N], acc_parts[1, :N], skip, Wlin, blin.reshape(1, 128))

    return node_embeddings, upd


# speedup vs baseline: 7.6021x; 1.0331x over previous
import functools

import jax
import jax.numpy as jnp
from jax import lax
from jax.experimental import pallas as pl
from jax.experimental.pallas import tpu as pltpu
from jax.experimental.pallas import tpu_sc as plsc

N = 10000
NP = 10112
E = 320000
B = 4096
NC = 2
NS = 16
NW = NC * NS

EV_CHUNK = 128
E_PER_TILE = E // NW
E_CHUNK = 32
E_TAIL = E_PER_TILE % E_CHUNK
R_CHUNK = 2000
SLABS = (128, 128, 128, 128, 120)

_mesh = plsc.VectorSubcoreMesh(core_axis_name="c", subcore_axis_name="s")
_cp = pltpu.CompilerParams(needs_layout_passes=False, use_tc_tiling_on_sc=False)


def _zero2d(ref, rows, cols):
    def body(i, _):
        for c in range(cols // 16):
            ref[i, pl.ds(c * 16, 16)] = jnp.zeros((16,), jnp.float32)
        return 0
    lax.fori_loop(0, rows, body, 0)


def _fill_table(zbuf, table, sid):
    for j, zr in enumerate(SLABS):
        pltpu.sync_copy(zbuf.at[pl.ds(0, zr)],
                        table.at[pl.ds(sid * 632 + j * 128, zr)])


def _dump_table(table, out, cid, sid):
    for j, zr in enumerate(SLABS):
        r0 = sid * 632 + j * 128
        pltpu.sync_copy(table.at[pl.ds(r0, zr)], out.at[cid, pl.ds(r0, zr)])


@functools.partial(
    pl.kernel, mesh=_mesh, compiler_params=_cp,
    out_type=[
        jax.ShapeDtypeStruct((NC, NP, 64), jnp.float32),
        jax.ShapeDtypeStruct((NC, NP, 64), jnp.float32),
    ],
    scratch_types=[
        pltpu.VMEM_SHARED((NP, 64), jnp.float32),
        pltpu.VMEM_SHARED((NP, 64), jnp.float32),
        pltpu.VMEM((EV_CHUNK, 64), jnp.float32),
        pltpu.VMEM((EV_CHUNK, 64), jnp.float32),
        pltpu.VMEM((EV_CHUNK,), jnp.int32),
        pltpu.VMEM((EV_CHUNK,), jnp.int32),
        pltpu.SemaphoreType.DMA,
        pltpu.SemaphoreType.DMA,
    ],
)
def _sc_a(src_ids, dst_ids, mem_hbm, o1, o2,
          s1, s2, smem_v, dmem_v, idxs_v, idxd_v, sem1, sem2):
    cid = lax.axis_index("c")
    sid = lax.axis_index("s")
    _zero2d(smem_v, EV_CHUNK, 64)
    _fill_table(smem_v, s1, sid)
    _fill_table(smem_v, s2, sid)
    plsc.subcore_barrier()

    ev0 = cid * (B // NC) + sid * EV_CHUNK
    pltpu.sync_copy(src_ids.at[pl.ds(ev0, EV_CHUNK)], idxs_v)
    pltpu.sync_copy(dst_ids.at[pl.ds(ev0, EV_CHUNK)], idxd_v)
    cp1 = pltpu.async_copy(mem_hbm.at[idxs_v], smem_v, sem1)
    cp2 = pltpu.async_copy(mem_hbm.at[idxd_v], dmem_v, sem2)
    cp1.wait()
    cp2.wait()
    pltpu.sync_copy(smem_v, s1.at[idxs_v], add=True)
    pltpu.sync_copy(smem_v, s1.at[idxd_v], add=True)
    pltpu.sync_copy(dmem_v, s2.at[idxs_v], add=True)
    pltpu.sync_copy(dmem_v, s2.at[idxd_v], add=True)

    plsc.subcore_barrier()
    _dump_table(s1, o1, cid, sid)
    _dump_table(s2, o2, cid, sid)


@functools.partial(
    pl.kernel, mesh=_mesh, compiler_params=_cp,
    out_type=[jax.ShapeDtypeStruct((NC, NP, 128), jnp.float32)],
    scratch_types=[
        pltpu.VMEM_SHARED((NP, 128), jnp.float32),
        pltpu.VMEM((EV_CHUNK, 128), jnp.float32),
        pltpu.VMEM((EV_CHUNK,), jnp.int32),
        pltpu.VMEM((EV_CHUNK,), jnp.int32),
    ],
)
def _sc_b(src_ids, dst_ids, eemb_hbm, o1, s1, ebuf_v, idxs_v, idxd_v):
    cid = lax.axis_index("c")
    sid = lax.axis_index("s")
    _zero2d(ebuf_v, EV_CHUNK, 128)
    _fill_table(ebuf_v, s1, sid)
    plsc.subcore_barrier()

    ev0 = cid * (B // NC) + sid * EV_CHUNK
    pltpu.sync_copy(src_ids.at[pl.ds(ev0, EV_CHUNK)], idxs_v)
    pltpu.sync_copy(dst_ids.at[pl.ds(ev0, EV_CHUNK)], idxd_v)
    pltpu.sync_copy(eemb_hbm.at[pl.ds(ev0, EV_CHUNK)], ebuf_v)
    pltpu.sync_copy(ebuf_v, s1.at[idxs_v], add=True)
    pltpu.sync_copy(ebuf_v, s1.at[idxd_v], add=True)

    plsc.subcore_barrier()
    _dump_table(s1, o1, cid, sid)


@functools.partial(
    pl.kernel, mesh=_mesh, compiler_params=_cp,
    out_type=[
        jax.ShapeDtypeStruct((NC, NP, 32), jnp.float32),
        jax.ShapeDtypeStruct((NC, NP, 16), jnp.float32),
        jax.ShapeDtypeStruct((E,), jnp.float32),
    ],
    scratch_types=[
        pltpu.VMEM_SHARED((NP, 32), jnp.float32),
        pltpu.VMEM_SHARED((NP, 16), jnp.float32),
        pltpu.VMEM((EV_CHUNK, 32), jnp.float32),
        pltpu.VMEM((EV_CHUNK, 16), jnp.float32),
        pltpu.VMEM((EV_CHUNK,), jnp.int32),
        pltpu.VMEM((EV_CHUNK,), jnp.int32),
        pltpu.VMEM((EV_CHUNK,), jnp.int32),
        pltpu.VMEM((112,), jnp.float32),
        pltpu.VMEM((B,), jnp.float32),
        pltpu.VMEM((R_CHUNK,), jnp.int32),
        pltpu.VMEM((N,), jnp.float32),
        pltpu.VMEM((R_CHUNK,), jnp.int32),
        pltpu.VMEM((R_CHUNK,), jnp.float32),
        pltpu.VMEM((R_CHUNK,), jnp.float32),
    ],
)
def _sc_c(src_ids, dst_ids, type_ids, time_embs, batch_hbm, ts_hbm,
          esrc_hbm, elu_hbm, embt_hbm,
          o1, o2, relt_out,
          s1, s2, tbuf_v, tybuf_v, idxs_v, idxd_v, tids_v, embt_v,
          tsv, bch_v, nts_v, sbuf_v, elub_v, rbuf_v):
    cid = lax.axis_index("c")
    sid = lax.axis_index("s")
    wid = sid * NC + cid
    _zero2d(tbuf_v, EV_CHUNK, 32)
    _fill_table(tbuf_v, s1, sid)
    _zero2d(tybuf_v, EV_CHUNK, 16)
    _fill_table(tybuf_v, s2, sid)
    plsc.subcore_barrier()

    pltpu.sync_copy(embt_hbm, embt_v)
    ev0 = cid * (B // NC) + sid * EV_CHUNK
    pltpu.sync_copy(src_ids.at[pl.ds(ev0, EV_CHUNK)], idxs_v)
    pltpu.sync_copy(dst_ids.at[pl.ds(ev0, EV_CHUNK)], idxd_v)
    pltpu.sync_copy(time_embs.at[pl.ds(ev0, EV_CHUNK)], tbuf_v)
    pltpu.sync_copy(type_ids.at[pl.ds(ev0, EV_CHUNK)], tids_v)
    lanes = lax.iota(jnp.int32, 16)

    def tyb(g, _):
        tv = tids_v[pl.ds(g * 16, 16)]
        for j in range(16):
            ti = tv[j]
            vals = plsc.load_gather(embt_v, [ti * 16 + lanes])
            tybuf_v[g * 16 + j, pl.ds(0, 16)] = vals
        return 0
    lax.fori_loop(0, EV_CHUNK // 16, tyb, 0)

    pltpu.sync_copy(tbuf_v, s1.at[idxs_v], add=True)
    pltpu.sync_copy(tbuf_v, s1.at[idxd_v], add=True)
    pltpu.sync_copy(tybuf_v, s2.at[idxs_v], add=True)
    pltpu.sync_copy(tybuf_v, s2.at[idxd_v], add=True)

    pltpu.sync_copy(ts_hbm, tsv)
    for m in range(N // R_CHUNK):
        pltpu.sync_copy(batch_hbm.at[pl.ds(m * R_CHUNK, R_CHUNK)], bch_v)

        def nbody(j, _):
            idx = bch_v[pl.ds(j * 16, 16)]
            nts_v[pl.ds(m * R_CHUNK + j * 16, 16)] = plsc.load_gather(tsv, [idx])
            return 0
        lax.fori_loop(0, R_CHUNK // 16, nbody, 0)

    for t in range(E_PER_TILE // R_CHUNK):
        eb = wid * E_PER_TILE + t * R_CHUNK
        pltpu.sync_copy(esrc_hbm.at[pl.ds(eb, R_CHUNK)], sbuf_v)
        pltpu.sync_copy(elu_hbm.at[pl.ds(eb, R_CHUNK)], elub_v)

        def rbody(j, _):
            ii = sbuf_v[pl.ds(j * 16, 16)]
            tse = plsc.load_gather(nts_v, [ii])
            rbuf_v[pl.ds(j * 16, 16)] = tse - elub_v[pl.ds(j * 16, 16)]
            return 0
        lax.fori_loop(0, R_CHUNK // 16, rbody, 0)
        pltpu.sync_copy(rbuf_v, relt_out.at[pl.ds(eb, R_CHUNK)])

    plsc.subcore_barrier()
    _dump_table(s1, o1, cid, sid)
    _dump_table(s2, o2, cid, sid)


E_BLK = 26
N_BLK = 12


@functools.partial(
    pl.kernel, mesh=_mesh, compiler_params=_cp,
    out_type=[jax.ShapeDtypeStruct((NC, NP, 144), jnp.float32)],
    scratch_types=[
        pltpu.VMEM_SHARED((NP, 144), jnp.float32),
        pltpu.VMEM((E_BLK * E_CHUNK,), jnp.int32),
        pltpu.VMEM((E_BLK, E_CHUNK), jnp.int32),
        pltpu.VMEM((2, E_CHUNK, 256), jnp.float32),
        pltpu.VMEM((2, E_CHUNK, 128), jnp.float32),
        pltpu.VMEM((2, E_CHUNK, 128), jnp.float32),
        pltpu.VMEM((E_CHUNK, 144), jnp.float32),
        pltpu.VMEM((E_CHUNK,), jnp.int32),
        pltpu.VMEM((1, E_CHUNK), jnp.int32),
        pltpu.SemaphoreType.DMA,
        pltpu.SemaphoreType.DMA,
    ],
)
def _sc_d(esrc_hbm, edst2_hbm, kv_hbm, q_hbm, etab_hbm,
          acc_out,
          acc, sblk_v, dblk_v, kvb_v, qb_v, eb_v, rowsb_v,
          tidxs_v, tdx_v, sem0, sem1):
    cid = lax.axis_index("c")
    sid = lax.axis_index("s")
    wid = sid * NC + cid
    sems = (sem0, sem1)

    _zero2d(rowsb_v, E_CHUNK, 144)
    nz = 632 // E_CHUNK
    for j in range(nz):
        pltpu.sync_copy(rowsb_v, acc.at[pl.ds(sid * 632 + j * E_CHUNK, E_CHUNK)])
    if 632 % E_CHUNK:
        pltpu.sync_copy(rowsb_v.at[pl.ds(0, 632 % E_CHUNK)],
                        acc.at[pl.ds(sid * 632 + nz * E_CHUNK, 632 % E_CHUNK)])
    plsc.subcore_barrier()

    inv_sqrt = jnp.float32(128.0 ** -0.5)
    lanes = lax.iota(jnp.int32, 16)
    umask = jnp.where(lanes == 0, jnp.float32(1.0), jnp.float32(0.0))

    def start(slot, brow, k):
        pltpu.async_copy(kv_hbm.at[sblk_v.at[pl.ds(k * E_CHUNK, E_CHUNK)]],
                         kvb_v.at[slot], sems[slot])
        pltpu.async_copy(q_hbm.at[dblk_v.at[k]], qb_v.at[slot], sems[slot])
        pltpu.async_copy(etab_hbm.at[pl.ds((brow + k) * E_CHUNK, E_CHUNK)],
                         eb_v.at[slot], sems[slot])

    def drain(slot, brow, k):
        pltpu.make_async_copy(kv_hbm.at[sblk_v.at[pl.ds(k * E_CHUNK, E_CHUNK)]],
                              kvb_v.at[slot], sems[slot]).wait()
        pltpu.make_async_copy(q_hbm.at[dblk_v.at[k]], qb_v.at[slot], sems[slot]).wait()
        pltpu.make_async_copy(etab_hbm.at[pl.ds((brow + k) * E_CHUNK, E_CHUNK)],
                              eb_v.at[slot], sems[slot]).wait()

    def compute(slot, idx_ref):
        def edge(i, _):
            d16 = jnp.zeros((16,), jnp.float32)
            for c in range(8):
                qv = qb_v[slot, i, pl.ds(c * 16, 16)]
                kvv = kvb_v[slot, i, pl.ds(c * 16, 16)]
                ev = eb_v[slot, i, pl.ds(c * 16, 16)]
                d16 = d16 + qv * (kvv + ev)
            s = jnp.sum(d16) * inv_sqrt
            svec = jnp.exp(jnp.full((16,), s, jnp.float32))
            for c in range(8):
                vv = kvb_v[slot, i, pl.ds(128 + c * 16, 16)]
                ev = eb_v[slot, i, pl.ds(c * 16, 16)]
                rowsb_v[i, pl.ds(c * 16, 16)] = svec * (vv + ev)
            rowsb_v[i, pl.ds(128, 16)] = svec * umask
            return 0
        lax.fori_loop(0, E_CHUNK, edge, 0)
        pltpu.sync_copy(rowsb_v, acc.at[idx_ref], add=True)

    def block(blk, _):
        brow = (wid * N_BLK + blk) * E_BLK
        pltpu.sync_copy(esrc_hbm.at[pl.ds(brow * E_CHUNK, E_BLK * E_CHUNK)], sblk_v)
        pltpu.sync_copy(edst2_hbm.at[pl.ds(brow, E_BLK)], dblk_v)
        start(0, brow, 0)

        def pair(j, _):
            start(1, brow, 2 * j + 1)
            drain(0, brow, 2 * j)
            compute(0, dblk_v.at[2 * j])

            @pl.when(2 * j + 2 < E_BLK)
            def _():
                start(0, brow, 2 * j + 2)
            drain(1, brow, 2 * j + 1)
            compute(1, dblk_v.at[2 * j + 1])
            return 0
        lax.fori_loop(0, E_BLK // 2, pair, 0)
        return 0
    lax.fori_loop(0, N_BLK, block, 0)

    @pl.when(wid < 16)
    def _():
        trow = NW * N_BLK * E_BLK + wid
        pltpu.sync_copy(esrc_hbm.at[pl.ds(trow * E_CHUNK, E_CHUNK)], tidxs_v)
        pltpu.sync_copy(edst2_hbm.at[pl.ds(trow, 1)], tdx_v)
        tc1 = pltpu.async_copy(kv_hbm.at[tidxs_v], kvb_v.at[0], sem0)
        tc2 = pltpu.async_copy(q_hbm.at[tdx_v.at[0]], qb_v.at[0], sem0)
        tc3 = pltpu.async_copy(etab_hbm.at[pl.ds(trow * E_CHUNK, E_CHUNK)],
                               eb_v.at[0], sem0)
        tc1.wait()
        tc2.wait()
        tc3.wait()
        compute(0, tdx_v.at[0])

    plsc.subcore_barrier()
    _dump_table(acc, acc_out, cid, sid)


def _tc0_body(ts_ref, wt_ref, bt_ref, out_ref):
    out_ref[...] = jnp.cos(ts_ref[...] * wt_ref[...] + bt_ref[...])


def _tcb_node_body(ms0, ms1, md0, md1, mt0, mt1, me0, me1, mty0, mty1,
                   mem_ref, x_ref, w1, w2, w3, w4, w6, bih, whh, bhh,
                   wq, wk, wv, wsk,
                   upd_ref, q_ref, kv_ref, skip_ref):
    f32 = jnp.float32
    ms = ms0[0] + ms1[0]
    md = md0[0] + md1[0]
    mt = mt0[0] + mt1[0]
    me = me0[0] + me1[0]
    mty = mty0[0] + mty1[0]
    cnt = mty[:, 8:9]
    inv = 1.0 / jnp.maximum(cnt, 1.0)
    gi = (jnp.dot(ms * inv, w1[...], preferred_element_type=f32)
          + jnp.dot(md * inv, w2[...], preferred_element_type=f32)
          + jnp.dot(mt * inv, w3[...], preferred_element_type=f32)
          + jnp.dot(me * inv, w4[...], preferred_element_type=f32)
          + jnp.dot(mty * inv, w6[...], preferred_element_type=f32)
          + bih[...])
    mem = mem_ref[...]
    gh = jnp.dot(mem, whh[...], preferred_element_type=f32) + bhh[...]
    d = 64
    r = jax.nn.sigmoid(gi[:, :d] + gh[:, :d])
    z = jax.nn.sigmoid(gi[:, d:2 * d] + gh[:, d:2 * d])
    n_ = jnp.tanh(gi[:, 2 * d:] + r * gh[:, 2 * d:])
    new_mem = (1.0 - z) * n_ + z * mem
    upd = jnp.where(cnt > 0, new_mem, mem)
    upd_ref[...] = upd
    x_cat = jnp.concatenate([x_ref[...], upd], axis=1)
    q_ref[...] = jnp.dot(x_cat, wq[...], preferred_element_type=f32)
    k = jnp.dot(x_cat, wk[...], preferred_element_type=f32)
    v = jnp.dot(x_cat, wv[...], preferred_element_type=f32)
    kv_ref[...] = jnp.concatenate([k, v], axis=1)
    skip_ref[...] = jnp.dot(x_cat, wsk[...], preferred_element_type=f32)


def _tcb_edge_body(rel_ref, attr_ref, wtc_ref, btc_ref, wet_ref, wea_ref, e_ref):
    cosvT = jnp.cos(wtc_ref[...] * rel_ref[0, 0] + btc_ref[...])
    e_ref[...] = (
        jax.lax.dot_general(cosvT, wet_ref[...], (((0,), (0,)), ((), ())),
                            preferred_element_type=jnp.float32)
        + jnp.dot(attr_ref[...], wea_ref[...],
                  preferred_element_type=jnp.float32))


def _tcd_body(a0_ref, a1_ref, skip_ref, wlin_ref, blin_ref, out_ref):
    acc = a0_ref[0] + a1_ref[0]
    denom = jnp.maximum(acc[:, 128:129], 1e-16)
    out = acc[:, :128] / denom + skip_ref[...]
    out_ref[...] = jnp.dot(out, wlin_ref[...],
                           preferred_element_type=jnp.float32) + blin_ref[...]


def kernel(event_type_ids, event_src_ids, event_dst_ids, event_embeddings,
           event_timestamps, x, edge_index, edge_attr, edge_last_update,
           batch, memory, emb_table, Wt, bt, W_ih, b_ih, W_hh, b_hh,
           Wq, Wk, Wv, We, Wskip, Wlin, blin):
    f32 = jnp.float32
    srcs = edge_index[0]
    dsts = edge_index[1]
    embt16 = jnp.concatenate([emb_table, jnp.ones((7, 1), f32),
                              jnp.zeros((7, 7), f32)], axis=1).reshape(112)
    W1 = W_ih[8:72]
    W2 = W_ih[72:136]
    W3 = W_ih[136:168]
    W4 = W_ih[168:296]
    W6 = jnp.concatenate([W_ih[0:8], jnp.zeros((8, 192), f32)], axis=0)
    WeT = We[:32]
    WeA = We[32:]

    time_embs = pl.pallas_call(
        _tc0_body,
        out_shape=jax.ShapeDtypeStruct((B, 32), f32),
    )(event_timestamps.reshape(B, 1), Wt, bt.reshape(1, 32))

    a1, a2 = _sc_a(event_src_ids, event_dst_ids, memory)
    (bb,) = _sc_b(event_src_ids, event_dst_ids, event_embeddings)
    c1, c2, rel_t = _sc_c(event_src_ids, event_dst_ids, event_type_ids,
                          time_embs, batch, event_timestamps, srcs,
                          edge_last_update, embt16)

    nb = 1000
    grid_n = N // nb
    row = lambda i: (i, 0)
    full = lambda i: (0, 0)
    c0 = lambda i: (0, i, 0)
    c1r = lambda i: (1, i, 0)
    upd, q, kv, skip = pl.pallas_call(
        _tcb_node_body,
        grid=(grid_n,),
        in_specs=[
            pl.BlockSpec((1, nb, 64), c0), pl.BlockSpec((1, nb, 64), c1r),
            pl.BlockSpec((1, nb, 64), c0), pl.BlockSpec((1, nb, 64), c1r),
            pl.BlockSpec((1, nb, 32), c0), pl.BlockSpec((1, nb, 32), c1r),
            pl.BlockSpec((1, nb, 128), c0), pl.BlockSpec((1, nb, 128), c1r),
            pl.BlockSpec((1, nb, 16), c0), pl.BlockSpec((1, nb, 16), c1r),
            pl.BlockSpec((nb, 64), row), pl.BlockSpec((nb, 128), row),
            pl.BlockSpec((64, 192), full), pl.BlockSpec((64, 192), full),
            pl.BlockSpec((32, 192), full), pl.BlockSpec((128, 192), full),
            pl.BlockSpec((16, 192), full),
            pl.BlockSpec((1, 192), full), pl.BlockSpec((64, 192), full),
            pl.BlockSpec((1, 192), full),
            pl.BlockSpec((192, 128), full), pl.BlockSpec((192, 128), full),
            pl.BlockSpec((192, 128), full), pl.BlockSpec((192, 128), full),
        ],
        out_specs=[
            pl.BlockSpec((nb, 64), row), pl.BlockSpec((nb, 128), row),
            pl.BlockSpec((nb, 256), row), pl.BlockSpec((nb, 128), row),
        ],
        out_shape=[
            jax.ShapeDtypeStruct((N, 64), f32),
            jax.ShapeDtypeStruct((N, 128), f32),
            jax.ShapeDtypeStruct((N, 256), f32),
            jax.ShapeDtypeStruct((N, 128), f32),
        ],
    )(a1, a1, a2, a2, c1, c1,
      bb, bb, c2, c2, memory, x,
      W1, W2, W3, W4, W6, b_ih.reshape(1, 192), W_hh, b_hh.reshape(1, 192),
      Wq, Wk, Wv, Wskip)

    ebk = 4000
    e_tab = pl.pallas_call(
        _tcb_edge_body,
        grid=(E // ebk,),
        in_specs=[
            pl.BlockSpec((1, 1, ebk), lambda i: (i, 0, 0)), pl.BlockSpec((ebk, 128), row),
            pl.BlockSpec((32, 1), full), pl.BlockSpec((32, 1), full),
            pl.BlockSpec((32, 128), full), pl.BlockSpec((128, 128), full),
        ],
        out_specs=pl.BlockSpec((ebk, 128), row),
        out_shape=jax.ShapeDtypeStruct((E, 128), f32),
    )(rel_t.reshape(E // ebk, 1, ebk), edge_attr, Wt.reshape(32, 1),
      bt.reshape(32, 1), WeT, WeA)

    (acc_parts,) = _sc_d(srcs, dsts.reshape(E // 32, 32), kv, q, e_tab)

    node_embeddings = pl.pallas_call(
        _tcd_body,
        grid=(grid_n,),
        in_specs=[
            pl.BlockSpec((1, nb, 144), c0), pl.BlockSpec((1, nb, 144), c1r),
            pl.BlockSpec((nb, 128), row),
            pl.BlockSpec((128, 128), full), pl.BlockSpec((1, 128), full),
        ],
        out_specs=pl.BlockSpec((nb, 128), row),
        out_shape=jax.ShapeDtypeStruct((N, 128), f32),
    )(acc_parts, acc_parts, skip, Wlin, blin.reshape(1, 128))

    return node_embeddings, upd


# speedup vs baseline: 7.6055x; 1.0004x over previous
import functools

import jax
import jax.numpy as jnp
from jax import lax
from jax.experimental import pallas as pl
from jax.experimental.pallas import tpu as pltpu
from jax.experimental.pallas import tpu_sc as plsc

N = 10000
NP = 10112
E = 320000
B = 4096
NC = 2
NS = 16
NW = NC * NS

EV_CHUNK = 128
E_PER_TILE = E // NW
E_CHUNK = 32
R_CHUNK = 2000
SLABS = (128, 128, 128, 128, 120)

_mesh = plsc.VectorSubcoreMesh(core_axis_name="c", subcore_axis_name="s")
_cp = pltpu.CompilerParams(needs_layout_passes=False, use_tc_tiling_on_sc=False)


def _zero2d(ref, rows, cols):
    def body(i, _):
        for c in range(cols // 16):
            ref[i, pl.ds(c * 16, 16)] = jnp.zeros((16,), jnp.float32)
        return 0
    lax.fori_loop(0, rows, body, 0)


def _fill_table(zbuf, table, sid):
    for j, zr in enumerate(SLABS):
        pltpu.sync_copy(zbuf.at[pl.ds(0, zr)],
                        table.at[pl.ds(sid * 632 + j * 128, zr)])


def _dump_table(table, out, cid, sid):
    for j, zr in enumerate(SLABS):
        r0 = sid * 632 + j * 128
        pltpu.sync_copy(table.at[pl.ds(r0, zr)], out.at[cid, pl.ds(r0, zr)])


@functools.partial(
    pl.kernel, mesh=_mesh, compiler_params=_cp,
    out_type=[
        jax.ShapeDtypeStruct((NC, NP, 64), jnp.float32),
        jax.ShapeDtypeStruct((NC, NP, 64), jnp.float32),
    ],
    scratch_types=[
        pltpu.VMEM_SHARED((NP, 64), jnp.float32),
        pltpu.VMEM_SHARED((NP, 64), jnp.float32),
        pltpu.VMEM((EV_CHUNK, 64), jnp.float32),
        pltpu.VMEM((EV_CHUNK, 64), jnp.float32),
        pltpu.VMEM((EV_CHUNK,), jnp.int32),
        pltpu.VMEM((EV_CHUNK,), jnp.int32),
        pltpu.SemaphoreType.DMA,
        pltpu.SemaphoreType.DMA,
    ],
)
def _sc_a(src_ids, dst_ids, mem_hbm, o1, o2,
          s1, s2, smem_v, dmem_v, idxs_v, idxd_v, sem1, sem2):
    cid = lax.axis_index("c")
    sid = lax.axis_index("s")
    _zero2d(smem_v, EV_CHUNK, 64)
    _fill_table(smem_v, s1, sid)
    _fill_table(smem_v, s2, sid)
    plsc.subcore_barrier()

    ev0 = cid * (B // NC) + sid * EV_CHUNK
    pltpu.sync_copy(src_ids.at[pl.ds(ev0, EV_CHUNK)], idxs_v)
    pltpu.sync_copy(dst_ids.at[pl.ds(ev0, EV_CHUNK)], idxd_v)
    cp1 = pltpu.async_copy(mem_hbm.at[idxs_v], smem_v, sem1)
    cp2 = pltpu.async_copy(mem_hbm.at[idxd_v], dmem_v, sem2)
    cp1.wait()
    cp2.wait()
    pltpu.sync_copy(smem_v, s1.at[idxs_v], add=True)
    pltpu.sync_copy(smem_v, s1.at[idxd_v], add=True)
    pltpu.sync_copy(dmem_v, s2.at[idxs_v], add=True)
    pltpu.sync_copy(dmem_v, s2.at[idxd_v], add=True)

    plsc.subcore_barrier()
    _dump_table(s1, o1, cid, sid)
    _dump_table(s2, o2, cid, sid)


@functools.partial(
    pl.kernel, mesh=_mesh, compiler_params=_cp,
    out_type=[jax.ShapeDtypeStruct((NC, NP, 128), jnp.float32)],
    scratch_types=[
        pltpu.VMEM_SHARED((NP, 128), jnp.float32),
        pltpu.VMEM((EV_CHUNK, 128), jnp.float32),
        pltpu.VMEM((EV_CHUNK,), jnp.int32),
        pltpu.VMEM((EV_CHUNK,), jnp.int32),
    ],
)
def _sc_b(src_ids, dst_ids, eemb_hbm, o1, s1, ebuf_v, idxs_v, idxd_v):
    cid = lax.axis_index("c")
    sid = lax.axis_index("s")
    _zero2d(ebuf_v, EV_CHUNK, 128)
    _fill_table(ebuf_v, s1, sid)
    plsc.subcore_barrier()

    ev0 = cid * (B // NC) + sid * EV_CHUNK
    pltpu.sync_copy(src_ids.at[pl.ds(ev0, EV_CHUNK)], idxs_v)
    pltpu.sync_copy(dst_ids.at[pl.ds(ev0, EV_CHUNK)], idxd_v)
    pltpu.sync_copy(eemb_hbm.at[pl.ds(ev0, EV_CHUNK)], ebuf_v)
    pltpu.sync_copy(ebuf_v, s1.at[idxs_v], add=True)
    pltpu.sync_copy(ebuf_v, s1.at[idxd_v], add=True)

    plsc.subcore_barrier()
    _dump_table(s1, o1, cid, sid)


@functools.partial(
    pl.kernel, mesh=_mesh, compiler_params=_cp,
    out_type=[
        jax.ShapeDtypeStruct((NC, NP, 32), jnp.float32),
        jax.ShapeDtypeStruct((NC, NP, 16), jnp.float32),
        jax.ShapeDtypeStruct((E,), jnp.float32),
    ],
    scratch_types=[
        pltpu.VMEM_SHARED((NP, 32), jnp.float32),
        pltpu.VMEM_SHARED((NP, 16), jnp.float32),
        pltpu.VMEM((EV_CHUNK, 32), jnp.float32),
        pltpu.VMEM((EV_CHUNK, 16), jnp.float32),
        pltpu.VMEM((EV_CHUNK,), jnp.int32),
        pltpu.VMEM((EV_CHUNK,), jnp.int32),
        pltpu.VMEM((EV_CHUNK,), jnp.int32),
        pltpu.VMEM((112,), jnp.float32),
        pltpu.VMEM((B,), jnp.float32),
        pltpu.VMEM((R_CHUNK,), jnp.int32),
        pltpu.VMEM((N,), jnp.float32),
        pltpu.VMEM((R_CHUNK,), jnp.int32),
        pltpu.VMEM((R_CHUNK,), jnp.float32),
        pltpu.VMEM((R_CHUNK,), jnp.float32),
    ],
)
def _sc_c(src_ids, dst_ids, type_ids, time_embs, batch_hbm, ts_hbm,
          esrc_hbm, elu_hbm, embt_hbm,
          o1, o2, relt_out,
          s1, s2, tbuf_v, tybuf_v, idxs_v, idxd_v, tids_v, embt_v,
          tsv, bch_v, nts_v, sbuf_v, elub_v, rbuf_v):
    cid = lax.axis_index("c")
    sid = lax.axis_index("s")
    wid = sid * NC + cid
    _zero2d(tbuf_v, EV_CHUNK, 32)
    _fill_table(tbuf_v, s1, sid)
    _zero2d(tybuf_v, EV_CHUNK, 16)
    _fill_table(tybuf_v, s2, sid)
    plsc.subcore_barrier()

    pltpu.sync_copy(embt_hbm, embt_v)
    ev0 = cid * (B // NC) + sid * EV_CHUNK
    pltpu.sync_copy(src_ids.at[pl.ds(ev0, EV_CHUNK)], idxs_v)
    pltpu.sync_copy(dst_ids.at[pl.ds(ev0, EV_CHUNK)], idxd_v)
    pltpu.sync_copy(time_embs.at[pl.ds(ev0, EV_CHUNK)], tbuf_v)
    pltpu.sync_copy(type_ids.at[pl.ds(ev0, EV_CHUNK)], tids_v)
    lanes = lax.iota(jnp.int32, 16)

    def tyb(g, _):
        tv = tids_v[pl.ds(g * 16, 16)]
        for j in range(16):
            ti = tv[j]
            vals = plsc.load_gather(embt_v, [ti * 16 + lanes])
            tybuf_v[g * 16 + j, pl.ds(0, 16)] = vals
        return 0
    lax.fori_loop(0, EV_CHUNK // 16, tyb, 0)

    pltpu.sync_copy(tbuf_v, s1.at[idxs_v], add=True)
    pltpu.sync_copy(tbuf_v, s1.at[idxd_v], add=True)
    pltpu.sync_copy(tybuf_v, s2.at[idxs_v], add=True)
    pltpu.sync_copy(tybuf_v, s2.at[idxd_v], add=True)

    pltpu.sync_copy(ts_hbm, tsv)
    for m in range(N // R_CHUNK):
        pltpu.sync_copy(batch_hbm.at[pl.ds(m * R_CHUNK, R_CHUNK)], bch_v)

        def nbody(j, _):
            idx = bch_v[pl.ds(j * 16, 16)]
            nts_v[pl.ds(m * R_CHUNK + j * 16, 16)] = plsc.load_gather(tsv, [idx])
            return 0
        lax.fori_loop(0, R_CHUNK // 16, nbody, 0)

    for t in range(E_PER_TILE // R_CHUNK):
        eb = wid * E_PER_TILE + t * R_CHUNK
        pltpu.sync_copy(esrc_hbm.at[pl.ds(eb, R_CHUNK)], sbuf_v)
        pltpu.sync_copy(elu_hbm.at[pl.ds(eb, R_CHUNK)], elub_v)

        def rbody(j, _):
            ii = sbuf_v[pl.ds(j * 16, 16)]
            tse = plsc.load_gather(nts_v, [ii])
            rbuf_v[pl.ds(j * 16, 16)] = tse - elub_v[pl.ds(j * 16, 16)]
            return 0
        lax.fori_loop(0, R_CHUNK // 16, rbody, 0)
        pltpu.sync_copy(rbuf_v, relt_out.at[pl.ds(eb, R_CHUNK)])

    plsc.subcore_barrier()
    _dump_table(s1, o1, cid, sid)
    _dump_table(s2, o2, cid, sid)


E_BLK = 26
N_BLK = 12


@functools.partial(
    pl.kernel, mesh=_mesh, compiler_params=_cp,
    out_type=[jax.ShapeDtypeStruct((NC, NP, 144), jnp.float32)],
    scratch_types=[
        pltpu.VMEM_SHARED((NP, 144), jnp.float32),
        pltpu.VMEM((E_BLK * E_CHUNK,), jnp.int32),
        pltpu.VMEM((E_BLK, E_CHUNK), jnp.int32),
        pltpu.VMEM((2, E_CHUNK, 256), jnp.float32),
        pltpu.VMEM((2, E_CHUNK, 128), jnp.float32),
        pltpu.VMEM((2, E_CHUNK, 128), jnp.float32),
        pltpu.VMEM((E_CHUNK, 144), jnp.float32),
        pltpu.VMEM((E_CHUNK,), jnp.int32),
        pltpu.VMEM((1, E_CHUNK), jnp.int32),
        pltpu.SemaphoreType.DMA,
        pltpu.SemaphoreType.DMA,
    ],
)
def _sc_d(esrc_hbm, edst2_hbm, kv_hbm, q_hbm, etab_hbm,
          acc_out,
          acc, sblk_v, dblk_v, kvb_v, qb_v, eb_v, rowsb_v,
          tidxs_v, tdx_v, sem0, sem1):
    cid = lax.axis_index("c")
    sid = lax.axis_index("s")
    wid = sid * NC + cid
    sems = (sem0, sem1)

    _zero2d(rowsb_v, E_CHUNK, 144)
    nz = 632 // E_CHUNK
    for j in range(nz):
        pltpu.sync_copy(rowsb_v, acc.at[pl.ds(sid * 632 + j * E_CHUNK, E_CHUNK)])
    if 632 % E_CHUNK:
        pltpu.sync_copy(rowsb_v.at[pl.ds(0, 632 % E_CHUNK)],
                        acc.at[pl.ds(sid * 632 + nz * E_CHUNK, 632 % E_CHUNK)])
    plsc.subcore_barrier()

    inv_sqrt = jnp.float32(128.0 ** -0.5)
    lanes = lax.iota(jnp.int32, 16)
    umask = jnp.where(lanes == 0, jnp.float32(1.0), jnp.float32(0.0))

    def start(slot, brow, k):
        pltpu.async_copy(kv_hbm.at[sblk_v.at[pl.ds(k * E_CHUNK, E_CHUNK)]],
                         kvb_v.at[slot], sems[slot])
        pltpu.async_copy(q_hbm.at[dblk_v.at[k]], qb_v.at[slot], sems[slot])
        pltpu.async_copy(etab_hbm.at[pl.ds((brow + k) * E_CHUNK, E_CHUNK)],
                         eb_v.at[slot], sems[slot])

    def drain(slot, brow, k):
        pltpu.make_async_copy(kv_hbm.at[sblk_v.at[pl.ds(k * E_CHUNK, E_CHUNK)]],
                              kvb_v.at[slot], sems[slot]).wait()
        pltpu.make_async_copy(q_hbm.at[dblk_v.at[k]], qb_v.at[slot], sems[slot]).wait()
        pltpu.make_async_copy(etab_hbm.at[pl.ds((brow + k) * E_CHUNK, E_CHUNK)],
                              eb_v.at[slot], sems[slot]).wait()

    def compute(slot, idx_ref):
        def edge(i, _):
            d16 = jnp.zeros((16,), jnp.float32)
            for c in range(8):
                qv = qb_v[slot, i, pl.ds(c * 16, 16)]
                kvv = kvb_v[slot, i, pl.ds(c * 16, 16)]
                ev = eb_v[slot, i, pl.ds(c * 16, 16)]
                d16 = d16 + qv * (kvv + ev)
            s = jnp.sum(d16) * inv_sqrt
            svec = jnp.exp(jnp.full((16,), s, jnp.float32))
            for c in range(8):
                vv = kvb_v[slot, i, pl.ds(128 + c * 16, 16)]
                ev = eb_v[slot, i, pl.ds(c * 16, 16)]
                rowsb_v[i, pl.ds(c * 16, 16)] = svec * (vv + ev)
            rowsb_v[i, pl.ds(128, 16)] = svec * umask
            return 0
        lax.fori_loop(0, E_CHUNK, edge, 0)
        pltpu.sync_copy(rowsb_v, acc.at[idx_ref], add=True)

    def block(blk, _):
        brow = (wid * N_BLK + blk) * E_BLK
        pltpu.sync_copy(esrc_hbm.at[pl.ds(brow * E_CHUNK, E_BLK * E_CHUNK)], sblk_v)
        pltpu.sync_copy(edst2_hbm.at[pl.ds(brow, E_BLK)], dblk_v)
        start(0, brow, 0)

        def pair(j, _):
            start(1, brow, 2 * j + 1)
            drain(0, brow, 2 * j)
            compute(0, dblk_v.at[2 * j])

            @pl.when(2 * j + 2 < E_BLK)
            def _():
                start(0, brow, 2 * j + 2)
            drain(1, brow, 2 * j + 1)
            compute(1, dblk_v.at[2 * j + 1])
            return 0
        lax.fori_loop(0, E_BLK // 2, pair, 0)
        return 0
    lax.fori_loop(0, N_BLK, block, 0)

    @pl.when(wid < 16)
    def _():
        trow = NW * N_BLK * E_BLK + wid
        pltpu.sync_copy(esrc_hbm.at[pl.ds(trow * E_CHUNK, E_CHUNK)], tidxs_v)
        pltpu.sync_copy(edst2_hbm.at[pl.ds(trow, 1)], tdx_v)
        tc1 = pltpu.async_copy(kv_hbm.at[tidxs_v], kvb_v.at[0], sem0)
        tc2 = pltpu.async_copy(q_hbm.at[tdx_v.at[0]], qb_v.at[0], sem0)
        tc3 = pltpu.async_copy(etab_hbm.at[pl.ds(trow * E_CHUNK, E_CHUNK)],
                               eb_v.at[0], sem0)
        tc1.wait()
        tc2.wait()
        tc3.wait()
        compute(0, tdx_v.at[0])

    plsc.subcore_barrier()
    _dump_table(acc, acc_out, cid, sid)


def _tc0_body(ts_ref, wt_ref, bt_ref, out_ref):
    out_ref[...] = jnp.cos(ts_ref[...] * wt_ref[...] + bt_ref[...])


def _tcb_node_body(ms0, ms1, md0, md1, mt0, mt1, me0, me1, mty0, mty1,
                   mem_ref, x_ref, w1, w2, w3, w4, w6, bih, whh, bhh,
                   wq, wk, wv, wsk,
                   upd_ref, q_ref, kv_ref, skip_ref):
    f32 = jnp.float32
    ms = ms0[0] + ms1[0]
    md = md0[0] + md1[0]
    mt = mt0[0] + mt1[0]
    me = me0[0] + me1[0]
    mty = mty0[0] + mty1[0]
    cnt = mty[:, 8:9]
    inv = 1.0 / jnp.maximum(cnt, 1.0)
    gi = (jnp.dot(ms * inv, w1[...], preferred_element_type=f32)
          + jnp.dot(md * inv, w2[...], preferred_element_type=f32)
          + jnp.dot(mt * inv, w3[...], preferred_element_type=f32)
          + jnp.dot(me * inv, w4[...], preferred_element_type=f32)
          + jnp.dot(mty * inv, w6[...], preferred_element_type=f32)
          + bih[...])
    mem = mem_ref[...]
    gh = jnp.dot(mem, whh[...], preferred_element_type=f32) + bhh[...]
    d = 64
    r = jax.nn.sigmoid(gi[:, :d] + gh[:, :d])
    z = jax.nn.sigmoid(gi[:, d:2 * d] + gh[:, d:2 * d])
    n_ = jnp.tanh(gi[:, 2 * d:] + r * gh[:, 2 * d:])
    new_mem = (1.0 - z) * n_ + z * mem
    upd = jnp.where(cnt > 0, new_mem, mem)
    upd_ref[...] = upd
    x_cat = jnp.concatenate([x_ref[...], upd], axis=1)
    q_ref[...] = jnp.dot(x_cat, wq[...], preferred_element_type=f32)
    k = jnp.dot(x_cat, wk[...], preferred_element_type=f32)
    v = jnp.dot(x_cat, wv[...], preferred_element_type=f32)
    kv_ref[...] = jnp.concatenate([k, v], axis=1)
    skip_ref[...] = jnp.dot(x_cat, wsk[...], preferred_element_type=f32)


def _tcb_edge_body(rel_ref, attr_ref, wtc_ref, btc_ref, wet_ref, wea_ref, e_ref):
    cosvT = jnp.cos(wtc_ref[...] * rel_ref[0, 0] + btc_ref[...])
    e_ref[...] = (
        jax.lax.dot_general(cosvT, wet_ref[...], (((0,), (0,)), ((), ())),
                            preferred_element_type=jnp.float32)
        + jnp.dot(attr_ref[...], wea_ref[...],
                  preferred_element_type=jnp.float32))


def _tcd_body(a0_ref, a1_ref, skip_ref, wlin_ref, blin_ref, out_ref):
    acc = a0_ref[0] + a1_ref[0]
    denom = jnp.maximum(acc[:, 128:129], 1e-16)
    out = acc[:, :128] / denom + skip_ref[...]
    out_ref[...] = jnp.dot(out, wlin_ref[...],
                           preferred_element_type=jnp.float32) + blin_ref[...]


def kernel(event_type_ids, event_src_ids, event_dst_ids, event_embeddings,
           event_timestamps, x, edge_index, edge_attr, edge_last_update,
           batch, memory, emb_table, Wt, bt, W_ih, b_ih, W_hh, b_hh,
           Wq, Wk, Wv, We, Wskip, Wlin, blin):
    f32 = jnp.float32
    srcs = edge_index[0]
    dsts = edge_index[1]
    embt16 = jnp.concatenate([emb_table, jnp.ones((7, 1), f32),
                              jnp.zeros((7, 7), f32)], axis=1).reshape(112)
    W1 = W_ih[8:72]
    W2 = W_ih[72:136]
    W3 = W_ih[136:168]
    W4 = W_ih[168:296]
    W6 = jnp.concatenate([W_ih[0:8], jnp.zeros((8, 192), f32)], axis=0)
    WeT = We[:32]
    WeA = We[32:]

    time_embs = pl.pallas_call(
        _tc0_body,
        out_shape=jax.ShapeDtypeStruct((B, 32), f32),
    )(event_timestamps.reshape(B, 1), Wt, bt.reshape(1, 32))

    a1, a2 = _sc_a(event_src_ids, event_dst_ids, memory)
    (bb,) = _sc_b(event_src_ids, event_dst_ids, event_embeddings)
    c1, c2, rel_t = _sc_c(event_src_ids, event_dst_ids, event_type_ids,
                          time_embs, batch, event_timestamps, srcs,
                          edge_last_update, embt16)

    nb = 1000
    grid_n = N // nb
    row = lambda i: (i, 0)
    full = lambda i: (0, 0)
    c0 = lambda i: (0, i, 0)
    c1r = lambda i: (1, i, 0)
    upd, q, kv, skip = pl.pallas_call(
        _tcb_node_body,
        grid=(grid_n,),
        in_specs=[
            pl.BlockSpec((1, nb, 64), c0), pl.BlockSpec((1, nb, 64), c1r),
            pl.BlockSpec((1, nb, 64), c0), pl.BlockSpec((1, nb, 64), c1r),
            pl.BlockSpec((1, nb, 32), c0), pl.BlockSpec((1, nb, 32), c1r),
            pl.BlockSpec((1, nb, 128), c0), pl.BlockSpec((1, nb, 128), c1r),
            pl.BlockSpec((1, nb, 16), c0), pl.BlockSpec((1, nb, 16), c1r),
            pl.BlockSpec((nb, 64), row), pl.BlockSpec((nb, 128), row),
            pl.BlockSpec((64, 192), full), pl.BlockSpec((64, 192), full),
            pl.BlockSpec((32, 192), full), pl.BlockSpec((128, 192), full),
            pl.BlockSpec((16, 192), full),
            pl.BlockSpec((1, 192), full), pl.BlockSpec((64, 192), full),
            pl.BlockSpec((1, 192), full),
            pl.BlockSpec((192, 128), full), pl.BlockSpec((192, 128), full),
            pl.BlockSpec((192, 128), full), pl.BlockSpec((192, 128), full),
        ],
        out_specs=[
            pl.BlockSpec((nb, 64), row), pl.BlockSpec((nb, 128), row),
            pl.BlockSpec((nb, 256), row), pl.BlockSpec((nb, 128), row),
        ],
        out_shape=[
            jax.ShapeDtypeStruct((N, 64), f32),
            jax.ShapeDtypeStruct((N, 128), f32),
            jax.ShapeDtypeStruct((N, 256), f32),
            jax.ShapeDtypeStruct((N, 128), f32),
        ],
    )(a1, a1, a2, a2, c1, c1,
      bb, bb, c2, c2, memory, x,
      W1, W2, W3, W4, W6, b_ih.reshape(1, 192), W_hh, b_hh.reshape(1, 192),
      Wq, Wk, Wv, Wskip)

    ebk = 4000
    e_tab = pl.pallas_call(
        _tcb_edge_body,
        grid=(E // ebk,),
        in_specs=[
            pl.BlockSpec((1, 1, ebk), lambda i: (i, 0, 0)), pl.BlockSpec((ebk, 128), row),
            pl.BlockSpec((32, 1), full), pl.BlockSpec((32, 1), full),
            pl.BlockSpec((32, 128), full), pl.BlockSpec((128, 128), full),
        ],
        out_specs=pl.BlockSpec((ebk, 128), row),
        out_shape=jax.ShapeDtypeStruct((E, 128), f32),
    )(rel_t.reshape(E // ebk, 1, ebk), edge_attr, Wt.reshape(32, 1),
      bt.reshape(32, 1), WeT, WeA)

    (acc_parts,) = _sc_d(srcs, dsts.reshape(E // 32, 32), kv, q, e_tab)

    node_embeddings = pl.pallas_call(
        _tcd_body,
        grid=(grid_n,),
        in_specs=[
            pl.BlockSpec((1, nb, 144), c0), pl.BlockSpec((1, nb, 144), c1r),
            pl.BlockSpec((nb, 128), row),
            pl.BlockSpec((128, 128), full), pl.BlockSpec((1, 128), full),
        ],
        out_specs=pl.BlockSpec((nb, 128), row),
        out_shape=jax.ShapeDtypeStruct((N, 128), f32),
    )(acc_parts, acc_parts, skip, Wlin, blin.reshape(1, 128))

    return node_embeddings, upd
